# Initial kernel scaffold; baseline (speedup 1.0000x reference)
import functools
import math

import jax
import jax.numpy as jnp
from jax import lax
from jax.experimental import pallas as pl
from jax.experimental.pallas import tpu as pltpu
from jax.experimental.pallas import tpu_sc as plsc

N = 10000
E = 160000
D = 256
ED = 16
TE = 64

N_PAD = 10240
E_PAD = 163840
NW = 32
EPW = E_PAD // NW
KCH = 128
NCH = EPW // KCH
NSUB = 16
ROWS_PER_TILE = N_PAD // NSUB
XW = 16

BN = 1024
BE = 512

_f32 = jnp.float32


def _silu(z):
  return z * jax.nn.sigmoid(z)


def _ln(h, g, b):
  mu = jnp.mean(h, axis=-1, keepdims=True)
  v = jnp.mean((h - mu) * (h - mu), axis=-1, keepdims=True)
  return (h - mu) * lax.rsqrt(v + 1e-5) * g + b


def _init_body(h_ref, new_ref, neb_ref, g_ref, b_ref, wa_ref, wb_ref,
               hn_ref, pa_ref, pb_ref):
  h0 = jnp.dot(h_ref[...], new_ref[...], preferred_element_type=_f32)
  h0 = h0 + neb_ref[...]
  hn = _ln(h0, g_ref[...], b_ref[...])
  hn_ref[...] = hn
  pa_ref[...] = jnp.dot(hn, wa_ref[...], preferred_element_type=_f32)
  pb_ref[...] = jnp.dot(hn, wb_ref[...], preferred_element_type=_f32)


def _edge_body(ga_ref, gb_ref, xr_ref, xc_ref, ea_ref, wea_ref, wr_ref,
               b1_ref, ew2_ref, b2_ref, aw_ref, ab_ref, cw1_ref, cb1_ref,
               cw2_ref, m_ref, cu_ref):
  cd = xr_ref[...] - xc_ref[...]
  radial = jnp.sum(cd * cd, axis=1, keepdims=True)
  e_in = (ga_ref[...] + gb_ref[...] + radial * wr_ref[...]
          + jnp.dot(ea_ref[...], wea_ref[...], preferred_element_type=_f32)
          + b1_ref[...])
  m1 = _silu(e_in)
  m2 = _silu(jnp.dot(m1, ew2_ref[...], preferred_element_type=_f32)
             + b2_ref[...])
  att = jax.nn.sigmoid(
      jnp.sum(m2 * aw_ref[...] + ab_ref[...], axis=1, keepdims=True))
  m = m2 * att
  t1 = _silu(jnp.dot(m, cw1_ref[...], preferred_element_type=_f32)
             + cb1_ref[...])
  cw = jnp.sum(t1 * cw2_ref[...], axis=1, keepdims=True)
  inv = lax.rsqrt(radial + 1e-8) * cw
  cu_ref[0] = cd * inv
  m_ref[0] = m[:, :128]
  m_ref[1] = m[:, 128:]


def _node_body(hn_ref, mi_ref, w1a_ref, w1bl_ref, w1bh_ref, nb1_ref,
               nw2_ref, nb2_ref, g_ref, b_ref, wa_ref, wb_ref,
               hn_out_ref, pa_ref, pb_ref):
  z = (jnp.dot(hn_ref[...], w1a_ref[...], preferred_element_type=_f32)
       + jnp.dot(mi_ref[0], w1bl_ref[...], preferred_element_type=_f32)
       + jnp.dot(mi_ref[1], w1bh_ref[...], preferred_element_type=_f32)
       + nb1_ref[...])
  hnew = hn_ref[...] + jnp.dot(_silu(z), nw2_ref[...],
                               preferred_element_type=_f32) + nb2_ref[...]
  hn2 = _ln(hnew, g_ref[...], b_ref[...])
  hn_out_ref[...] = hn2
  pa_ref[...] = jnp.dot(hn2, wa_ref[...], preferred_element_type=_f32)
  pb_ref[...] = jnp.dot(hn2, wb_ref[...], preferred_element_type=_f32)


def _final_body(hn_ref, mi_ref, w1a_ref, w1bl_ref, w1bh_ref, nb1_ref,
                nw2_ref, nb2_ref, o1_ref, o1b_ref, o2_ref, o2b_ref,
                chw_ref, chb_ref, x4_ref, x0_ref, ho_ref, xo_ref):
  z = (jnp.dot(hn_ref[...], w1a_ref[...], preferred_element_type=_f32)
       + jnp.dot(mi_ref[0], w1bl_ref[...], preferred_element_type=_f32)
       + jnp.dot(mi_ref[1], w1bh_ref[...], preferred_element_type=_f32)
       + nb1_ref[...])
  h4 = hn_ref[...] + jnp.dot(_silu(z), nw2_ref[...],
                             preferred_element_type=_f32) + nb2_ref[...]
  ho = _silu(jnp.dot(h4, o1_ref[...], preferred_element_type=_f32)
             + o1b_ref[...])
  ho_ref[...] = jnp.dot(ho, o2_ref[...], preferred_element_type=_f32) \
      + o2b_ref[...]
  ch = jnp.dot(h4, chw_ref[...], preferred_element_type=_f32) + chb_ref[...]
  xo_ref[...] = x4_ref[...] - x0_ref[...] + ch


def _row_spec(bn, w):
  return pl.BlockSpec((bn, w), lambda i: (i, 0))


def _full_spec(shape):
  nd = len(shape)
  return pl.BlockSpec(shape, lambda i: (0,) * nd)


_NODE_GRID = (N_PAD // BN,)
_EDGE_GRID = (E_PAD // BE,)
_TC_PARAMS = pltpu.CompilerParams(dimension_semantics=("arbitrary",))

_init_call = pl.pallas_call(
    _init_body,
    grid=_NODE_GRID,
    in_specs=[_row_spec(BN, D)] + [_full_spec((D, D))] + [_full_spec((1, D))] * 3
    + [_full_spec((D, D))] * 2,
    out_specs=[_row_spec(BN, D)] * 3,
    out_shape=[jax.ShapeDtypeStruct((N_PAD, D), _f32)] * 3,
    compiler_params=_TC_PARAMS,
)

_edge_call = pl.pallas_call(
    _edge_body,
    grid=_EDGE_GRID,
    in_specs=[
        _row_spec(BE, D), _row_spec(BE, D),
        _row_spec(BE, XW), _row_spec(BE, XW),
        _row_spec(BE, ED), _full_spec((ED, D)),
        _full_spec((1, D)), _full_spec((1, D)),
        _full_spec((D, D)), _full_spec((1, D)),
        _full_spec((1, D)), _full_spec((1, D)),
        _full_spec((D, D)), _full_spec((1, D)),
        _full_spec((1, D)),
    ],
    out_specs=[
        pl.BlockSpec((2, BE, 128), lambda i: (0, i, 0)),
        pl.BlockSpec((1, BE, XW), lambda i: (0, i, 0)),
    ],
    out_shape=[
        jax.ShapeDtypeStruct((2, E_PAD, 128), _f32),
        jax.ShapeDtypeStruct((1, E_PAD, XW), _f32),
    ],
    compiler_params=_TC_PARAMS,
)

_node_call = pl.pallas_call(
    _node_body,
    grid=_NODE_GRID,
    in_specs=[
        _row_spec(BN, D),
        pl.BlockSpec((2, BN, 128), lambda i: (0, i, 0)),
        _full_spec((D, D)), _full_spec((128, D)), _full_spec((128, D)),
        _full_spec((1, D)), _full_spec((D, D)), _full_spec((1, D)),
        _full_spec((1, D)), _full_spec((1, D)),
        _full_spec((D, D)), _full_spec((D, D)),
    ],
    out_specs=[_row_spec(BN, D)] * 3,
    out_shape=[jax.ShapeDtypeStruct((N_PAD, D), _f32)] * 3,
    compiler_params=_TC_PARAMS,
)

_final_call = pl.pallas_call(
    _final_body,
    grid=_NODE_GRID,
    in_specs=[
        _row_spec(BN, D),
        pl.BlockSpec((2, BN, 128), lambda i: (0, i, 0)),
        _full_spec((D, D)), _full_spec((128, D)), _full_spec((128, D)),
        _full_spec((1, D)), _full_spec((D, D)), _full_spec((1, D)),
        _full_spec((D, D)), _full_spec((1, D)),
        _full_spec((D, D)), _full_spec((1, D)),
        _full_spec((D, XW)), _full_spec((1, XW)),
        _row_spec(BN, XW), _row_spec(BN, XW),
    ],
    out_specs=[_row_spec(BN, D), _row_spec(BN, XW)],
    out_shape=[
        jax.ShapeDtypeStruct((N_PAD, D), _f32),
        jax.ShapeDtypeStruct((N_PAD, XW), _f32),
    ],
    compiler_params=_TC_PARAMS,
)


_SC_MESH = plsc.VectorSubcoreMesh(core_axis_name="c", subcore_axis_name="s")


@functools.partial(
    pl.kernel,
    out_type=[
        jax.ShapeDtypeStruct((E_PAD, D), _f32),
        jax.ShapeDtypeStruct((E_PAD, D), _f32),
        jax.ShapeDtypeStruct((E_PAD, XW), _f32),
        jax.ShapeDtypeStruct((E_PAD, XW), _f32),
    ],
    mesh=_SC_MESH,
    scratch_types=[
        pltpu.VMEM((NCH, KCH), jnp.int32),
        pltpu.VMEM((NCH, KCH), jnp.int32),
        pltpu.VMEM((KCH, D), _f32),
        pltpu.VMEM((KCH, D), _f32),
        pltpu.VMEM((KCH, XW), _f32),
        pltpu.VMEM((KCH, XW), _f32),
        pltpu.SemaphoreType.DMA,
        pltpu.SemaphoreType.DMA,
        pltpu.SemaphoreType.DMA,
        pltpu.SemaphoreType.DMA,
    ],
)
def _gather_call(pa_hbm, pb_hbm, xp_hbm, ri_hbm, ci_hbm,
                 ga_hbm, gb_hbm, xr_hbm, xc_hbm,
                 ri_v, ci_v, ba, bb, bxr, bxc, sa, sb, sxr, sxc):
  c = lax.axis_index("c")
  s = lax.axis_index("s")
  wid = s * 2 + c
  pltpu.sync_copy(ri_hbm.at[wid], ri_v)
  pltpu.sync_copy(ci_hbm.at[wid], ci_v)
  base = wid * EPW

  def body(j, carry):
    e0 = base + j * KCH
    da = pltpu.async_copy(pa_hbm.at[ri_v.at[j]], ba, sa)
    db = pltpu.async_copy(pb_hbm.at[ci_v.at[j]], bb, sb)
    dr = pltpu.async_copy(xp_hbm.at[ri_v.at[j]], bxr, sxr)
    dc = pltpu.async_copy(xp_hbm.at[ci_v.at[j]], bxc, sxc)
    da.wait()
    db.wait()
    dr.wait()
    dc.wait()
    pltpu.sync_copy(ba, ga_hbm.at[pl.ds(e0, KCH)])
    pltpu.sync_copy(bb, gb_hbm.at[pl.ds(e0, KCH)])
    pltpu.sync_copy(bxr, xr_hbm.at[pl.ds(e0, KCH)])
    pltpu.sync_copy(bxc, xc_hbm.at[pl.ds(e0, KCH)])
    return carry

  lax.fori_loop(0, NCH, body, 0)


@functools.partial(
    pl.kernel,
    out_type=[
        jax.ShapeDtypeStruct((2, N_PAD, 128), _f32),
        jax.ShapeDtypeStruct((N_PAD, XW), _f32),
    ],
    mesh=_SC_MESH,
    scratch_types=[
        pltpu.VMEM((NCH, KCH), jnp.int32),
        pltpu.VMEM((KCH, 128), _f32),
        pltpu.VMEM((KCH, XW), _f32),
        pltpu.VMEM_SHARED((N_PAD, 128), _f32),
        pltpu.VMEM_SHARED((N_PAD, XW), _f32),
    ],
)
def _scatter_call(m2_hbm, cu_hbm, ri_hbm, x_hbm, zer_hbm,
                  mi_hbm, xn_hbm,
                  ri_v, bm, bcu, acc_m, acc_x):
  c = lax.axis_index("c")
  s = lax.axis_index("s")
  r0 = s * ROWS_PER_TILE
  pltpu.sync_copy(zer_hbm.at[pl.ds(r0, ROWS_PER_TILE)],
                  acc_m.at[pl.ds(r0, ROWS_PER_TILE)])

  @pl.when(c == 0)
  def _():
    pltpu.sync_copy(x_hbm.at[pl.ds(r0, ROWS_PER_TILE)],
                    acc_x.at[pl.ds(r0, ROWS_PER_TILE)])

  plsc.subcore_barrier()
  wid = s * 2 + c
  pltpu.sync_copy(ri_hbm.at[wid], ri_v)
  base = wid * EPW

  def body(j, carry):
    e0 = base + j * KCH
    pltpu.sync_copy(m2_hbm.at[c, pl.ds(e0, KCH)], bm)
    pltpu.sync_copy(bm, acc_m.at[ri_v.at[j]], add=True)

    @pl.when(c == 0)
    def _():
      pltpu.sync_copy(cu_hbm.at[0, pl.ds(e0, KCH)], bcu)
      pltpu.sync_copy(bcu, acc_x.at[ri_v.at[j]], add=True)

    return carry

  lax.fori_loop(0, NCH, body, 0)
  plsc.subcore_barrier()
  pltpu.sync_copy(acc_m.at[pl.ds(r0, ROWS_PER_TILE)],
                  mi_hbm.at[c, pl.ds(r0, ROWS_PER_TILE)])

  @pl.when(c == 0)
  def _():
    pltpu.sync_copy(acc_x.at[pl.ds(r0, ROWS_PER_TILE)],
                    xn_hbm.at[pl.ds(r0, ROWS_PER_TILE)])


def _time_row(t, params):
  half = TE // 2
  e = math.log(10000.0) / (half - 1)
  freqs = jnp.exp(jnp.arange(half, dtype=_f32) * -e)
  a = t[:, None] * freqs[None, :]
  te = jnp.concatenate([jnp.sin(a), jnp.cos(a)], axis=-1)
  te = _silu(te @ params['t1_w'] + params['t1_b'])
  te = te @ params['t2_w'] + params['t2_b']
  return te[0:1]


def _row(v):
  return v.reshape(1, -1)


def kernel(h, x, edge_attr, t, params, edge_index):
  hpad = jnp.pad(h, ((0, N_PAD - N), (0, 0)))
  xpad = jnp.pad(x, ((0, N_PAD - N), (0, XW - 3)))
  eapad = jnp.pad(edge_attr, ((0, E_PAD - E), (0, 0)))
  npad_e = E_PAD - E
  pad_idx = N + (jnp.arange(npad_e, dtype=jnp.int32) % 64)
  ri = jnp.concatenate([edge_index[0], pad_idx]).reshape(NW, NCH, KCH)
  ci = jnp.concatenate([edge_index[1], pad_idx]).reshape(NW, NCH, KCH)
  zer = jnp.zeros((N_PAD, 128), _f32)

  neb = _row(params['ne_b']) + _time_row(t, params)
  l0 = params['layers'][0]
  hn, pa, pb = _init_call(
      hpad, params['ne_w'], neb, _row(l0['ln_g']), _row(l0['ln_b']),
      l0['e_w1'][:D], l0['e_w1'][D:2 * D])

  xcur = xpad
  out = None
  for li in range(4):
    lp = params['layers'][li]
    ga, gb, xr, xc = _gather_call(pa, pb, xcur, ri, ci)
    m2, cu = _edge_call(
        ga, gb, xr, xc, eapad,
        lp['e_w1'][2 * D + 1:], lp['e_w1'][2 * D:2 * D + 1], _row(lp['e_b1']),
        lp['e_w2'], _row(lp['e_b2']),
        _row(lp['a_w'][:, 0]), jnp.full((1, D), lp['a_b'][0] / D, _f32),
        lp['c_w1'], _row(lp['c_b1']), _row(lp['c_w2'][:, 0]))
    mi, xnew = _scatter_call(m2, cu, ri, xcur, zer)
    xcur = xnew
    if li < 3:
      lnx = params['layers'][li + 1]
      hn, pa, pb = _node_call(
          hn, mi,
          lp['n_w1'][:D], lp['n_w1'][D:D + 128], lp['n_w1'][D + 128:],
          _row(lp['n_b1']), lp['n_w2'], _row(lp['n_b2']),
          _row(lnx['ln_g']), _row(lnx['ln_b']),
          lnx['e_w1'][:D], lnx['e_w1'][D:2 * D])
    else:
      chw = jnp.pad(params['ch_w'], ((0, 0), (0, XW - 3)))
      chb = jnp.pad(params['ch_b'], (0, XW - 3)).reshape(1, XW)
      ho, xo = _final_call(
          hn, mi,
          lp['n_w1'][:D], lp['n_w1'][D:D + 128], lp['n_w1'][D + 128:],
          _row(lp['n_b1']), lp['n_w2'], _row(lp['n_b2']),
          params['o1_w'], _row(params['o1_b']),
          params['o2_w'], _row(params['o2_b']),
          chw, chb, xcur, xpad)
      out = (ho[:N], xo[:N, :3])
  return out


# speedup vs baseline: 2.0565x; 2.0565x over previous
import functools
import math

import jax
import jax.numpy as jnp
from jax import lax
from jax.experimental import pallas as pl
from jax.experimental.pallas import tpu as pltpu
from jax.experimental.pallas import tpu_sc as plsc

N = 10000
E = 160000
D = 256
ED = 16
TE = 64

N_PAD = 10240
E_PAD = 163840
NW = 32
EPW = E_PAD // NW
KCH = 128
NCH = EPW // KCH
NSUB = 16
ROWS_PER_TILE = N_PAD // NSUB
XW = 16

BN = 1024
BE = 512

_f32 = jnp.float32


def _silu(z):
  return z * jax.nn.sigmoid(z)


def _ln(h, g, b):
  mu = jnp.mean(h, axis=-1, keepdims=True)
  v = jnp.mean((h - mu) * (h - mu), axis=-1, keepdims=True)
  return (h - mu) * lax.rsqrt(v + 1e-5) * g + b


def _init_body(h_ref, new_ref, neb_ref, g_ref, b_ref, wa_ref, wb_ref,
               hn_ref, pa_ref, pb_ref):
  h0 = jnp.dot(h_ref[...], new_ref[...], preferred_element_type=_f32)
  h0 = h0 + neb_ref[...]
  hn = _ln(h0, g_ref[...], b_ref[...])
  hn_ref[...] = hn
  pa_ref[...] = jnp.dot(hn, wa_ref[...], preferred_element_type=_f32)
  pb_ref[...] = jnp.dot(hn, wb_ref[...], preferred_element_type=_f32)


def _edge_body(ga_ref, gb_ref, xr_ref, xc_ref, ea_ref, wea_ref, wr_ref,
               b1_ref, ew2_ref, b2_ref, aw_ref, ab_ref, cw1_ref, cb1_ref,
               cw2_ref, m_ref, cu_ref):
  cd = xr_ref[...] - xc_ref[...]
  radial = jnp.sum(cd * cd, axis=1, keepdims=True)
  e_in = (ga_ref[...] + gb_ref[...] + radial * wr_ref[...]
          + jnp.dot(ea_ref[...], wea_ref[...], preferred_element_type=_f32)
          + b1_ref[...])
  m1 = _silu(e_in)
  m2 = _silu(jnp.dot(m1, ew2_ref[...], preferred_element_type=_f32)
             + b2_ref[...])
  att = jax.nn.sigmoid(
      jnp.sum(m2 * aw_ref[...] + ab_ref[...], axis=1, keepdims=True))
  m = m2 * att
  t1 = _silu(jnp.dot(m, cw1_ref[...], preferred_element_type=_f32)
             + cb1_ref[...])
  cw = jnp.sum(t1 * cw2_ref[...], axis=1, keepdims=True)
  inv = lax.rsqrt(radial + 1e-8) * cw
  cu_ref[0] = cd * inv
  m_ref[0] = m[:, :128]
  m_ref[1] = m[:, 128:]


def _node_body(hn_ref, mi_ref, xp_ref, xt_ref, w1a_ref, w1bl_ref, w1bh_ref,
               nb1_ref, nw2_ref, nb2_ref, g_ref, b_ref, wa_ref, wb_ref,
               hn_out_ref, pa_ref, pb_ref, xt_out_ref):
  z = (jnp.dot(hn_ref[...], w1a_ref[...], preferred_element_type=_f32)
       + jnp.dot(mi_ref[0], w1bl_ref[...], preferred_element_type=_f32)
       + jnp.dot(mi_ref[1], w1bh_ref[...], preferred_element_type=_f32)
       + nb1_ref[...])
  hnew = hn_ref[...] + jnp.dot(_silu(z), nw2_ref[...],
                               preferred_element_type=_f32) + nb2_ref[...]
  hn2 = _ln(hnew, g_ref[...], b_ref[...])
  hn_out_ref[...] = hn2
  pa_ref[...] = jnp.dot(hn2, wa_ref[...], preferred_element_type=_f32)
  pb_ref[...] = jnp.dot(hn2, wb_ref[...], preferred_element_type=_f32)
  xt_out_ref[...] = xt_ref[...] + xp_ref[0] + xp_ref[1]


def _final_body(hn_ref, mi_ref, xp_ref, xt_ref, w1a_ref, w1bl_ref, w1bh_ref,
                nb1_ref, nw2_ref, nb2_ref, o1_ref, o1b_ref, o2_ref, o2b_ref,
                chw_ref, chb_ref, x0_ref, ho_ref, xo_ref):
  z = (jnp.dot(hn_ref[...], w1a_ref[...], preferred_element_type=_f32)
       + jnp.dot(mi_ref[0], w1bl_ref[...], preferred_element_type=_f32)
       + jnp.dot(mi_ref[1], w1bh_ref[...], preferred_element_type=_f32)
       + nb1_ref[...])
  h4 = hn_ref[...] + jnp.dot(_silu(z), nw2_ref[...],
                             preferred_element_type=_f32) + nb2_ref[...]
  ho = _silu(jnp.dot(h4, o1_ref[...], preferred_element_type=_f32)
             + o1b_ref[...])
  ho_ref[...] = jnp.dot(ho, o2_ref[...], preferred_element_type=_f32) \
      + o2b_ref[...]
  x4 = xt_ref[...] + xp_ref[0] + xp_ref[1]
  ch = jnp.dot(h4, chw_ref[...], preferred_element_type=_f32) + chb_ref[...]
  xo_ref[...] = x4 - x0_ref[...] + ch


def _row_spec(bn, w):
  return pl.BlockSpec((bn, w), lambda i: (i, 0))


def _full_spec(shape):
  nd = len(shape)
  return pl.BlockSpec(shape, lambda i: (0,) * nd)


_NODE_GRID = (N_PAD // BN,)
_EDGE_GRID = (E_PAD // BE,)
_TC_PARAMS = pltpu.CompilerParams(dimension_semantics=("arbitrary",))

_init_call = pl.pallas_call(
    _init_body,
    grid=_NODE_GRID,
    in_specs=[_row_spec(BN, D)] + [_full_spec((D, D))] + [_full_spec((1, D))] * 3
    + [_full_spec((D, D))] * 2,
    out_specs=[_row_spec(BN, D)] * 3,
    out_shape=[jax.ShapeDtypeStruct((N_PAD, D), _f32)] * 3,
    compiler_params=_TC_PARAMS,
)

_edge_call = pl.pallas_call(
    _edge_body,
    grid=_EDGE_GRID,
    in_specs=[
        _row_spec(BE, D), _row_spec(BE, D),
        _row_spec(BE, XW), _row_spec(BE, XW),
        _row_spec(BE, ED), _full_spec((ED, D)),
        _full_spec((1, D)), _full_spec((1, D)),
        _full_spec((D, D)), _full_spec((1, D)),
        _full_spec((1, D)), _full_spec((1, D)),
        _full_spec((D, D)), _full_spec((1, D)),
        _full_spec((1, D)),
    ],
    out_specs=[
        pl.BlockSpec((2, BE, 128), lambda i: (0, i, 0)),
        pl.BlockSpec((1, BE, XW), lambda i: (0, i, 0)),
    ],
    out_shape=[
        jax.ShapeDtypeStruct((2, E_PAD, 128), _f32),
        jax.ShapeDtypeStruct((1, E_PAD, XW), _f32),
    ],
    compiler_params=_TC_PARAMS,
)

_node_call = pl.pallas_call(
    _node_body,
    grid=_NODE_GRID,
    in_specs=[
        _row_spec(BN, D),
        pl.BlockSpec((2, BN, 128), lambda i: (0, i, 0)),
        pl.BlockSpec((2, BN, 128), lambda i: (0, i, 0)),
        _row_spec(BN, 128),
        _full_spec((D, D)), _full_spec((128, D)), _full_spec((128, D)),
        _full_spec((1, D)), _full_spec((D, D)), _full_spec((1, D)),
        _full_spec((1, D)), _full_spec((1, D)),
        _full_spec((D, D)), _full_spec((D, D)),
    ],
    out_specs=[_row_spec(BN, D)] * 3 + [_row_spec(BN, 128)],
    out_shape=[jax.ShapeDtypeStruct((N_PAD, D), _f32)] * 3
    + [jax.ShapeDtypeStruct((N_PAD, 128), _f32)],
    compiler_params=_TC_PARAMS,
)

_final_call = pl.pallas_call(
    _final_body,
    grid=_NODE_GRID,
    in_specs=[
        _row_spec(BN, D),
        pl.BlockSpec((2, BN, 128), lambda i: (0, i, 0)),
        pl.BlockSpec((2, BN, 128), lambda i: (0, i, 0)),
        _row_spec(BN, 128),
        _full_spec((D, D)), _full_spec((128, D)), _full_spec((128, D)),
        _full_spec((1, D)), _full_spec((D, D)), _full_spec((1, D)),
        _full_spec((D, D)), _full_spec((1, D)),
        _full_spec((D, D)), _full_spec((1, D)),
        _full_spec((D, 128)), _full_spec((1, 128)),
        _row_spec(BN, 128),
    ],
    out_specs=[_row_spec(BN, D), _row_spec(BN, 128)],
    out_shape=[
        jax.ShapeDtypeStruct((N_PAD, D), _f32),
        jax.ShapeDtypeStruct((N_PAD, 128), _f32),
    ],
    compiler_params=_TC_PARAMS,
)


_SC_MESH = plsc.VectorSubcoreMesh(
    core_axis_name="c", subcore_axis_name="s", num_cores=2, num_subcores=NSUB)


@functools.partial(
    pl.kernel,
    out_type=[
        jax.ShapeDtypeStruct((E_PAD, D), _f32),
        jax.ShapeDtypeStruct((E_PAD, D), _f32),
    ],
    mesh=_SC_MESH,
    scratch_types=[
        pltpu.VMEM((NCH, KCH), jnp.int32),
        pltpu.VMEM((NCH, KCH), jnp.int32),
        pltpu.VMEM((KCH, D), _f32),
        pltpu.VMEM((KCH, D), _f32),
        pltpu.SemaphoreType.DMA,
        pltpu.SemaphoreType.DMA,
    ],
)
def _gather_call(pa_hbm, pb_hbm, ri_hbm, ci_hbm, ga_hbm, gb_hbm,
                 ri_v, ci_v, ba, bb, sa, sb):
  c = lax.axis_index("c")
  s = lax.axis_index("s")
  wid = s * 2 + c
  pltpu.sync_copy(ri_hbm.at[wid], ri_v)
  pltpu.sync_copy(ci_hbm.at[wid], ci_v)
  base = wid * EPW

  def body(j, carry):
    e0 = base + j * KCH
    da = pltpu.async_copy(pa_hbm.at[ri_v.at[j]], ba, sa)
    db = pltpu.async_copy(pb_hbm.at[ci_v.at[j]], bb, sb)
    da.wait()
    db.wait()
    pltpu.sync_copy(ba, ga_hbm.at[pl.ds(e0, KCH)])
    pltpu.sync_copy(bb, gb_hbm.at[pl.ds(e0, KCH)])
    return carry

  lax.fori_loop(0, NCH, body, 0)


@functools.partial(
    pl.kernel,
    out_type=[
        jax.ShapeDtypeStruct((E_PAD, XW), _f32),
        jax.ShapeDtypeStruct((E_PAD, XW), _f32),
    ],
    mesh=_SC_MESH,
    scratch_types=[
        pltpu.VMEM((NCH, KCH), jnp.int32),
        pltpu.VMEM((NCH, KCH), jnp.int32),
        pltpu.VMEM((KCH, 128), _f32),
        pltpu.VMEM((KCH, 128), _f32),
        pltpu.VMEM((KCH, XW), _f32),
        pltpu.VMEM((KCH, XW), _f32),
        pltpu.SemaphoreType.DMA,
        pltpu.SemaphoreType.DMA,
    ],
)
def _gatherx_call(xt_hbm, ri_hbm, ci_hbm, xr_hbm, xc_hbm,
                  ri_v, ci_v, bxr, bxc, bxrs, bxcs, sr, sc_):
  c = lax.axis_index("c")
  s = lax.axis_index("s")
  wid = s * 2 + c
  pltpu.sync_copy(ri_hbm.at[wid], ri_v)
  pltpu.sync_copy(ci_hbm.at[wid], ci_v)
  base = wid * EPW

  def body(j, carry):
    e0 = base + j * KCH
    dr = pltpu.async_copy(xt_hbm.at[ri_v.at[j]], bxr, sr)
    dc = pltpu.async_copy(xt_hbm.at[ci_v.at[j]], bxc, sc_)
    dr.wait()
    dc.wait()

    def comp(i, cc):
      bxrs[i, :] = bxr[i, pl.ds(0, XW)]
      bxcs[i, :] = bxc[i, pl.ds(0, XW)]
      return cc

    lax.fori_loop(0, KCH, comp, 0)
    pltpu.sync_copy(bxrs, xr_hbm.at[pl.ds(e0, KCH)])
    pltpu.sync_copy(bxcs, xc_hbm.at[pl.ds(e0, KCH)])
    return carry

  lax.fori_loop(0, NCH, body, 0)


@functools.partial(
    pl.kernel,
    out_type=jax.ShapeDtypeStruct((2, N_PAD, 128), _f32),
    mesh=_SC_MESH,
    scratch_types=[
        pltpu.VMEM((NCH, KCH), jnp.int32),
        pltpu.VMEM((KCH, 128), _f32),
        pltpu.VMEM_SHARED((N_PAD, 128), _f32),
    ],
)
def _scatterm_call(m2_hbm, ri_hbm, zer_hbm, mi_hbm, ri_v, bm, acc_m):
  c = lax.axis_index("c")
  s = lax.axis_index("s")
  r0 = s * ROWS_PER_TILE
  pltpu.sync_copy(zer_hbm.at[pl.ds(r0, ROWS_PER_TILE)],
                  acc_m.at[pl.ds(r0, ROWS_PER_TILE)])
  plsc.subcore_barrier()
  for k in range(2):
    wid = s * 2 + k
    pltpu.sync_copy(ri_hbm.at[wid], ri_v)
    base = wid * EPW

    def body(j, carry):
      e0 = base + j * KCH
      pltpu.sync_copy(m2_hbm.at[c, pl.ds(e0, KCH)], bm)
      pltpu.sync_copy(bm, acc_m.at[ri_v.at[j]], add=True)
      return carry

    lax.fori_loop(0, NCH, body, 0)
  plsc.subcore_barrier()
  pltpu.sync_copy(acc_m.at[pl.ds(r0, ROWS_PER_TILE)],
                  mi_hbm.at[c, pl.ds(r0, ROWS_PER_TILE)])


@functools.partial(
    pl.kernel,
    out_type=jax.ShapeDtypeStruct((2, N_PAD, 128), _f32),
    mesh=_SC_MESH,
    scratch_types=[
        pltpu.VMEM((NCH, KCH), jnp.int32),
        pltpu.VMEM((KCH, XW), _f32),
        pltpu.VMEM((KCH, 128), _f32),
        pltpu.VMEM_SHARED((N_PAD, 128), _f32),
    ],
)
def _scatterx_call(cu_hbm, ri_hbm, zer_hbm, xp_hbm, ri_v, bcu, bcw, acc_x):
  c = lax.axis_index("c")
  s = lax.axis_index("s")
  r0 = s * ROWS_PER_TILE
  pltpu.sync_copy(zer_hbm.at[pl.ds(r0, ROWS_PER_TILE)],
                  acc_x.at[pl.ds(r0, ROWS_PER_TILE)])

  def zrow(i, cc):
    bcw[lax.div(i, 8), pl.ds(lax.rem(i, 8) * XW, XW)] = jnp.zeros((XW,), _f32)
    return cc

  lax.fori_loop(0, KCH * 8, zrow, 0)
  plsc.subcore_barrier()
  wid = c * NSUB + s
  pltpu.sync_copy(ri_hbm.at[wid], ri_v)
  base = wid * EPW

  def body(j, carry):
    e0 = base + j * KCH
    pltpu.sync_copy(cu_hbm.at[0, pl.ds(e0, KCH)], bcu)

    def widen(i, cc):
      bcw[i, pl.ds(0, XW)] = bcu[i, :]
      return cc

    lax.fori_loop(0, KCH, widen, 0)
    pltpu.sync_copy(bcw, acc_x.at[ri_v.at[j]], add=True)
    return carry

  lax.fori_loop(0, NCH, body, 0)
  plsc.subcore_barrier()
  pltpu.sync_copy(acc_x.at[pl.ds(r0, ROWS_PER_TILE)],
                  xp_hbm.at[c, pl.ds(r0, ROWS_PER_TILE)])


def _time_row(t, params):
  half = TE // 2
  e = math.log(10000.0) / (half - 1)
  freqs = jnp.exp(jnp.arange(half, dtype=_f32) * -e)
  a = t[:, None] * freqs[None, :]
  te = jnp.concatenate([jnp.sin(a), jnp.cos(a)], axis=-1)
  te = _silu(te @ params['t1_w'] + params['t1_b'])
  te = te @ params['t2_w'] + params['t2_b']
  return te[0:1]


def _row(v):
  return v.reshape(1, -1)


def kernel(h, x, edge_attr, t, params, edge_index):
  hpad = jnp.pad(h, ((0, N_PAD - N), (0, 0)))
  xtab = jnp.pad(x, ((0, N_PAD - N), (0, 128 - 3)))
  eapad = jnp.pad(edge_attr, ((0, E_PAD - E), (0, 0)))
  npad_e = E_PAD - E
  pad_idx = N + (jnp.arange(npad_e, dtype=jnp.int32) % 64)
  ri = jnp.concatenate([edge_index[0], pad_idx]).reshape(NW, NCH, KCH)
  ci = jnp.concatenate([edge_index[1], pad_idx]).reshape(NW, NCH, KCH)
  zer = jnp.zeros((N_PAD, 128), _f32)

  neb = _row(params['ne_b']) + _time_row(t, params)
  l0 = params['layers'][0]
  hn, pa, pb = _init_call(
      hpad, params['ne_w'], neb, _row(l0['ln_g']), _row(l0['ln_b']),
      l0['e_w1'][:D], l0['e_w1'][D:2 * D])

  out = None
  for li in range(4):
    lp = params['layers'][li]
    ga, gb = _gather_call(pa, pb, ri, ci)
    xr, xc = _gatherx_call(xtab, ri, ci)
    m2, cu = _edge_call(
        ga, gb, xr, xc, eapad,
        lp['e_w1'][2 * D + 1:], lp['e_w1'][2 * D:2 * D + 1], _row(lp['e_b1']),
        lp['e_w2'], _row(lp['e_b2']),
        _row(lp['a_w'][:, 0]), jnp.full((1, D), lp['a_b'][0] / D, _f32),
        lp['c_w1'], _row(lp['c_b1']), _row(lp['c_w2'][:, 0]))
    mi = _scatterm_call(m2, ri, zer)
    xp = _scatterx_call(cu, ri, zer)
    if li < 3:
      lnx = params['layers'][li + 1]
      hn, pa, pb, xtab = _node_call(
          hn, mi, xp, xtab,
          lp['n_w1'][:D], lp['n_w1'][D:D + 128], lp['n_w1'][D + 128:],
          _row(lp['n_b1']), lp['n_w2'], _row(lp['n_b2']),
          _row(lnx['ln_g']), _row(lnx['ln_b']),
          lnx['e_w1'][:D], lnx['e_w1'][D:2 * D])
    else:
      chw = jnp.pad(params['ch_w'], ((0, 0), (0, 128 - 3)))
      chb = jnp.pad(params['ch_b'], (0, 128 - 3)).reshape(1, 128)
      x0tab = jnp.pad(x, ((0, N_PAD - N), (0, 128 - 3)))
      ho, xo = _final_call(
          hn, mi, xp, xtab,
          lp['n_w1'][:D], lp['n_w1'][D:D + 128], lp['n_w1'][D + 128:],
          _row(lp['n_b1']), lp['n_w2'], _row(lp['n_b2']),
          params['o1_w'], _row(params['o1_b']),
          params['o2_w'], _row(params['o2_b']),
          chw, chb, x0tab)
      out = (ho[:N], xo[:N, :3])
  return out


# speedup vs baseline: 2.3336x; 1.1348x over previous
import functools
import math

import jax
import jax.numpy as jnp
from jax import lax
from jax.experimental import pallas as pl
from jax.experimental.pallas import tpu as pltpu
from jax.experimental.pallas import tpu_sc as plsc

N = 10000
E = 160000
D = 256
ED = 16
TE = 64

N_PAD = 10240
E_PAD = 163840
NW = 32
EPW = E_PAD // NW
KCH = 128
NCH = EPW // KCH
NSUB = 16
ROWS_PER_TILE = N_PAD // NSUB
XW = 16

BN = 1024
BE = 512

_f32 = jnp.float32


def _silu(z):
  return z * jax.nn.sigmoid(z)


def _ln(h, g, b):
  mu = jnp.mean(h, axis=-1, keepdims=True)
  v = jnp.mean((h - mu) * (h - mu), axis=-1, keepdims=True)
  return (h - mu) * lax.rsqrt(v + 1e-5) * g + b


def _init_body(h_ref, new_ref, neb_ref, g_ref, b_ref, wa_ref, wb_ref,
               hn_ref, pa_ref, pb_ref):
  h0 = jnp.dot(h_ref[...], new_ref[...], preferred_element_type=_f32)
  h0 = h0 + neb_ref[...]
  hn = _ln(h0, g_ref[...], b_ref[...])
  hn_ref[...] = hn
  pa_ref[...] = jnp.dot(hn, wa_ref[...], preferred_element_type=_f32)
  pb_ref[...] = jnp.dot(hn, wb_ref[...], preferred_element_type=_f32)


def _edge_body(ga_ref, gb_ref, xr_ref, xc_ref, ea_ref, wea_ref, wr_ref,
               b1_ref, ew2_ref, b2_ref, aw_ref, ab_ref, cw1_ref, cb1_ref,
               cw2_ref, m_ref, cu_ref):
  cd = xr_ref[...] - xc_ref[...]
  radial = jnp.sum(cd * cd, axis=1, keepdims=True)
  e_in = (ga_ref[...] + gb_ref[...] + radial * wr_ref[...]
          + jnp.dot(ea_ref[...], wea_ref[...], preferred_element_type=_f32)
          + b1_ref[...])
  m1 = _silu(e_in)
  m2 = _silu(jnp.dot(m1, ew2_ref[...], preferred_element_type=_f32)
             + b2_ref[...])
  att = jax.nn.sigmoid(
      jnp.sum(m2 * aw_ref[...] + ab_ref[...], axis=1, keepdims=True))
  m = m2 * att
  t1 = _silu(jnp.dot(m, cw1_ref[...], preferred_element_type=_f32)
             + cb1_ref[...])
  cw = jnp.sum(t1 * cw2_ref[...], axis=1, keepdims=True)
  inv = lax.rsqrt(radial + 1e-8) * cw
  cu_ref[0] = cd * inv
  m_ref[0] = m[:, :128]
  m_ref[1] = m[:, 128:]


def _node_body(hn_ref, mi_ref, xp_ref, xt_ref, w1a_ref, w1bl_ref, w1bh_ref,
               nb1_ref, nw2_ref, nb2_ref, g_ref, b_ref, wa_ref, wb_ref,
               hn_out_ref, pa_ref, pb_ref, xt_out_ref):
  z = (jnp.dot(hn_ref[...], w1a_ref[...], preferred_element_type=_f32)
       + jnp.dot(mi_ref[0], w1bl_ref[...], preferred_element_type=_f32)
       + jnp.dot(mi_ref[1], w1bh_ref[...], preferred_element_type=_f32)
       + nb1_ref[...])
  hnew = hn_ref[...] + jnp.dot(_silu(z), nw2_ref[...],
                               preferred_element_type=_f32) + nb2_ref[...]
  hn2 = _ln(hnew, g_ref[...], b_ref[...])
  hn_out_ref[...] = hn2
  pa_ref[...] = jnp.dot(hn2, wa_ref[...], preferred_element_type=_f32)
  pb_ref[...] = jnp.dot(hn2, wb_ref[...], preferred_element_type=_f32)
  xt_out_ref[...] = xt_ref[...] + xp_ref[0] + xp_ref[1]


def _final_body(hn_ref, mi_ref, xp_ref, xt_ref, w1a_ref, w1bl_ref, w1bh_ref,
                nb1_ref, nw2_ref, nb2_ref, o1_ref, o1b_ref, o2_ref, o2b_ref,
                chw_ref, chb_ref, x0_ref, ho_ref, xo_ref):
  z = (jnp.dot(hn_ref[...], w1a_ref[...], preferred_element_type=_f32)
       + jnp.dot(mi_ref[0], w1bl_ref[...], preferred_element_type=_f32)
       + jnp.dot(mi_ref[1], w1bh_ref[...], preferred_element_type=_f32)
       + nb1_ref[...])
  h4 = hn_ref[...] + jnp.dot(_silu(z), nw2_ref[...],
                             preferred_element_type=_f32) + nb2_ref[...]
  ho = _silu(jnp.dot(h4, o1_ref[...], preferred_element_type=_f32)
             + o1b_ref[...])
  ho_ref[...] = jnp.dot(ho, o2_ref[...], preferred_element_type=_f32) \
      + o2b_ref[...]
  x4 = xt_ref[...] + xp_ref[0] + xp_ref[1]
  ch = jnp.dot(h4, chw_ref[...], preferred_element_type=_f32) + chb_ref[...]
  xo_ref[...] = x4 - x0_ref[...] + ch


def _row_spec(bn, w):
  return pl.BlockSpec((bn, w), lambda i: (i, 0))


def _full_spec(shape):
  nd = len(shape)
  return pl.BlockSpec(shape, lambda i: (0,) * nd)


_NODE_GRID = (N_PAD // BN,)
_EDGE_GRID = (E_PAD // BE,)
_TC_PARAMS = pltpu.CompilerParams(dimension_semantics=("arbitrary",))

_init_call = pl.pallas_call(
    _init_body,
    grid=_NODE_GRID,
    in_specs=[_row_spec(BN, D)] + [_full_spec((D, D))] + [_full_spec((1, D))] * 3
    + [_full_spec((D, D))] * 2,
    out_specs=[_row_spec(BN, D)] * 3,
    out_shape=[jax.ShapeDtypeStruct((N_PAD, D), _f32)] * 3,
    compiler_params=_TC_PARAMS,
)

_edge_call = pl.pallas_call(
    _edge_body,
    grid=_EDGE_GRID,
    in_specs=[
        _row_spec(BE, D), _row_spec(BE, D),
        _row_spec(BE, XW), _row_spec(BE, XW),
        _row_spec(BE, ED), _full_spec((ED, D)),
        _full_spec((1, D)), _full_spec((1, D)),
        _full_spec((D, D)), _full_spec((1, D)),
        _full_spec((1, D)), _full_spec((1, D)),
        _full_spec((D, D)), _full_spec((1, D)),
        _full_spec((1, D)),
    ],
    out_specs=[
        pl.BlockSpec((2, BE, 128), lambda i: (0, i, 0)),
        pl.BlockSpec((1, BE, XW), lambda i: (0, i, 0)),
    ],
    out_shape=[
        jax.ShapeDtypeStruct((2, E_PAD, 128), _f32),
        jax.ShapeDtypeStruct((1, E_PAD, XW), _f32),
    ],
    compiler_params=_TC_PARAMS,
)

_node_call = pl.pallas_call(
    _node_body,
    grid=_NODE_GRID,
    in_specs=[
        _row_spec(BN, D),
        pl.BlockSpec((2, BN, 128), lambda i: (0, i, 0)),
        pl.BlockSpec((2, BN, 128), lambda i: (0, i, 0)),
        _row_spec(BN, 128),
        _full_spec((D, D)), _full_spec((128, D)), _full_spec((128, D)),
        _full_spec((1, D)), _full_spec((D, D)), _full_spec((1, D)),
        _full_spec((1, D)), _full_spec((1, D)),
        _full_spec((D, D)), _full_spec((D, D)),
    ],
    out_specs=[_row_spec(BN, D)] * 3 + [_row_spec(BN, 128)],
    out_shape=[jax.ShapeDtypeStruct((N_PAD, D), _f32)] * 3
    + [jax.ShapeDtypeStruct((N_PAD, 128), _f32)],
    compiler_params=_TC_PARAMS,
)

_final_call = pl.pallas_call(
    _final_body,
    grid=_NODE_GRID,
    in_specs=[
        _row_spec(BN, D),
        pl.BlockSpec((2, BN, 128), lambda i: (0, i, 0)),
        pl.BlockSpec((2, BN, 128), lambda i: (0, i, 0)),
        _row_spec(BN, 128),
        _full_spec((D, D)), _full_spec((128, D)), _full_spec((128, D)),
        _full_spec((1, D)), _full_spec((D, D)), _full_spec((1, D)),
        _full_spec((D, D)), _full_spec((1, D)),
        _full_spec((D, D)), _full_spec((1, D)),
        _full_spec((D, 128)), _full_spec((1, 128)),
        _row_spec(BN, 128),
    ],
    out_specs=[_row_spec(BN, D), _row_spec(BN, 128)],
    out_shape=[
        jax.ShapeDtypeStruct((N_PAD, D), _f32),
        jax.ShapeDtypeStruct((N_PAD, 128), _f32),
    ],
    compiler_params=_TC_PARAMS,
)


_SC_MESH = plsc.VectorSubcoreMesh(
    core_axis_name="c", subcore_axis_name="s", num_cores=2, num_subcores=NSUB)


KG = 64
NG = EPW // KG


@functools.partial(
    pl.kernel,
    out_type=[
        jax.ShapeDtypeStruct((E_PAD, D), _f32),
        jax.ShapeDtypeStruct((E_PAD, D), _f32),
    ],
    mesh=_SC_MESH,
    scratch_types=[
        pltpu.VMEM((NG, KG), jnp.int32),
        pltpu.VMEM((NG, KG), jnp.int32),
        [pltpu.VMEM((KG, D), _f32)] * 2,
        [pltpu.VMEM((KG, D), _f32)] * 2,
        [pltpu.SemaphoreType.DMA] * 2,
        [pltpu.SemaphoreType.DMA] * 2,
    ],
)
def _gather_call(pa_hbm, pb_hbm, ri_hbm, ci_hbm, ga_hbm, gb_hbm,
                 ri_v, ci_v, ba, bb, sa, sb):
  c = lax.axis_index("c")
  s = lax.axis_index("s")
  wid = s * 2 + c
  pltpu.sync_copy(ri_hbm.at[wid], ri_v)
  pltpu.sync_copy(ci_hbm.at[wid], ci_v)
  base = wid * EPW

  def start(j, k):
    pltpu.async_copy(pa_hbm.at[ri_v.at[j]], ba[k], sa[k])
    pltpu.async_copy(pb_hbm.at[ci_v.at[j]], bb[k], sb[k])

  def finish(j, k):
    pltpu.make_async_copy(pa_hbm.at[ri_v.at[j]], ba[k], sa[k]).wait()
    pltpu.make_async_copy(pb_hbm.at[ci_v.at[j]], bb[k], sb[k]).wait()
    e0 = base + j * KG
    pltpu.sync_copy(ba[k], ga_hbm.at[pl.ds(e0, KG)])
    pltpu.sync_copy(bb[k], gb_hbm.at[pl.ds(e0, KG)])

  start(0, 0)

  def body(g, carry):
    j0 = g * 2
    start(j0 + 1, 1)
    finish(j0, 0)

    @pl.when(g < NG // 2 - 1)
    def _():
      start(j0 + 2, 0)

    finish(j0 + 1, 1)
    return carry

  lax.fori_loop(0, NG // 2, body, 0)


@functools.partial(
    pl.kernel,
    out_type=[
        jax.ShapeDtypeStruct((E_PAD, XW), _f32),
        jax.ShapeDtypeStruct((E_PAD, XW), _f32),
    ],
    mesh=_SC_MESH,
    scratch_types=[
        pltpu.VMEM((NG, KG), jnp.int32),
        pltpu.VMEM((NG, KG), jnp.int32),
        [pltpu.VMEM((KG, 128), _f32)] * 2,
        [pltpu.VMEM((KG, 128), _f32)] * 2,
        pltpu.VMEM((KG, XW), _f32),
        pltpu.VMEM((KG, XW), _f32),
        [pltpu.SemaphoreType.DMA] * 2,
        [pltpu.SemaphoreType.DMA] * 2,
    ],
)
def _gatherx_call(xt_hbm, ri_hbm, ci_hbm, xr_hbm, xc_hbm,
                  ri_v, ci_v, bxr, bxc, bxrs, bxcs, sr, sc_):
  c = lax.axis_index("c")
  s = lax.axis_index("s")
  wid = s * 2 + c
  pltpu.sync_copy(ri_hbm.at[wid], ri_v)
  pltpu.sync_copy(ci_hbm.at[wid], ci_v)
  base = wid * EPW

  def start(j, k):
    pltpu.async_copy(xt_hbm.at[ri_v.at[j]], bxr[k], sr[k])
    pltpu.async_copy(xt_hbm.at[ci_v.at[j]], bxc[k], sc_[k])

  def finish(j, k):
    pltpu.make_async_copy(xt_hbm.at[ri_v.at[j]], bxr[k], sr[k]).wait()
    pltpu.make_async_copy(xt_hbm.at[ci_v.at[j]], bxc[k], sc_[k]).wait()

    def comp(i, cc, k=k):
      bxrs[i, :] = bxr[k][i, pl.ds(0, XW)]
      bxcs[i, :] = bxc[k][i, pl.ds(0, XW)]
      return cc

    lax.fori_loop(0, KG, comp, 0)
    e0 = base + j * KG
    pltpu.sync_copy(bxrs, xr_hbm.at[pl.ds(e0, KG)])
    pltpu.sync_copy(bxcs, xc_hbm.at[pl.ds(e0, KG)])

  start(0, 0)

  def body(g, carry):
    j0 = g * 2
    start(j0 + 1, 1)
    finish(j0, 0)

    @pl.when(g < NG // 2 - 1)
    def _():
      start(j0 + 2, 0)

    finish(j0 + 1, 1)
    return carry

  lax.fori_loop(0, NG // 2, body, 0)


@functools.partial(
    pl.kernel,
    out_type=jax.ShapeDtypeStruct((2, N_PAD, 128), _f32),
    mesh=_SC_MESH,
    scratch_types=[
        pltpu.VMEM((2 * NCH, KCH), jnp.int32),
        [pltpu.VMEM((KCH, 128), _f32)] * 2,
        [pltpu.SemaphoreType.DMA] * 2,
        pltpu.VMEM_SHARED((N_PAD, 128), _f32),
    ],
)
def _scatterm_call(m2_hbm, ri_hbm, zer_hbm, mi_hbm, ri_v, bm, sm, acc_m):
  c = lax.axis_index("c")
  s = lax.axis_index("s")
  r0 = s * ROWS_PER_TILE
  pltpu.sync_copy(zer_hbm.at[pl.ds(r0, ROWS_PER_TILE)],
                  acc_m.at[pl.ds(r0, ROWS_PER_TILE)])
  plsc.subcore_barrier()
  pltpu.sync_copy(ri_hbm.at[s * 2], ri_v.at[pl.ds(0, NCH)])
  pltpu.sync_copy(ri_hbm.at[s * 2 + 1], ri_v.at[pl.ds(NCH, NCH)])
  base = s * 2 * EPW

  def start(j, k):
    pltpu.async_copy(m2_hbm.at[c, pl.ds(base + j * KCH, KCH)], bm[k], sm[k])

  def finish(j, k):
    pltpu.make_async_copy(
        m2_hbm.at[c, pl.ds(base + j * KCH, KCH)], bm[k], sm[k]).wait()
    pltpu.sync_copy(bm[k], acc_m.at[ri_v.at[j]], add=True)

  start(0, 0)

  def body(g, carry):
    j0 = g * 2
    start(j0 + 1, 1)
    finish(j0, 0)

    @pl.when(g < NCH - 1)
    def _():
      start(j0 + 2, 0)

    finish(j0 + 1, 1)
    return carry

  lax.fori_loop(0, NCH, body, 0)
  plsc.subcore_barrier()
  pltpu.sync_copy(acc_m.at[pl.ds(r0, ROWS_PER_TILE)],
                  mi_hbm.at[c, pl.ds(r0, ROWS_PER_TILE)])


@functools.partial(
    pl.kernel,
    out_type=jax.ShapeDtypeStruct((2, N_PAD, 128), _f32),
    mesh=_SC_MESH,
    scratch_types=[
        pltpu.VMEM((NG, KG), jnp.int32),
        [pltpu.VMEM((KG, XW), _f32)] * 2,
        pltpu.VMEM((KG, 128), _f32),
        [pltpu.SemaphoreType.DMA] * 2,
        pltpu.VMEM_SHARED((N_PAD, 128), _f32),
    ],
)
def _scatterx_call(cu_hbm, ri_hbm, zer_hbm, xp_hbm, ri_v, bcu, bcw, scu,
                   acc_x):
  c = lax.axis_index("c")
  s = lax.axis_index("s")
  r0 = s * ROWS_PER_TILE
  pltpu.sync_copy(zer_hbm.at[pl.ds(r0, ROWS_PER_TILE)],
                  acc_x.at[pl.ds(r0, ROWS_PER_TILE)])

  def zrow(i, cc):
    bcw[lax.div(i, 8), pl.ds(lax.rem(i, 8) * XW, XW)] = jnp.zeros((XW,), _f32)
    return cc

  lax.fori_loop(0, KG * 8, zrow, 0)
  plsc.subcore_barrier()
  wid = c * NSUB + s
  pltpu.sync_copy(ri_hbm.at[wid], ri_v)
  base = wid * EPW

  def start(j, k):
    pltpu.async_copy(cu_hbm.at[0, pl.ds(base + j * KG, KG)], bcu[k], scu[k])

  def finish(j, k):
    pltpu.make_async_copy(cu_hbm.at[0, pl.ds(base + j * KG, KG)],
                          bcu[k], scu[k]).wait()

    def widen(i, cc, k=k):
      bcw[i, pl.ds(0, XW)] = bcu[k][i, :]
      return cc

    lax.fori_loop(0, KG, widen, 0)
    pltpu.sync_copy(bcw, acc_x.at[ri_v.at[j]], add=True)

  start(0, 0)

  def body(g, carry):
    j0 = g * 2
    start(j0 + 1, 1)
    finish(j0, 0)

    @pl.when(g < NG // 2 - 1)
    def _():
      start(j0 + 2, 0)

    finish(j0 + 1, 1)
    return carry

  lax.fori_loop(0, NG // 2, body, 0)
  plsc.subcore_barrier()
  pltpu.sync_copy(acc_x.at[pl.ds(r0, ROWS_PER_TILE)],
                  xp_hbm.at[c, pl.ds(r0, ROWS_PER_TILE)])


def _time_row(t, params):
  half = TE // 2
  e = math.log(10000.0) / (half - 1)
  freqs = jnp.exp(jnp.arange(half, dtype=_f32) * -e)
  a = t[:, None] * freqs[None, :]
  te = jnp.concatenate([jnp.sin(a), jnp.cos(a)], axis=-1)
  te = _silu(te @ params['t1_w'] + params['t1_b'])
  te = te @ params['t2_w'] + params['t2_b']
  return te[0:1]


def _row(v):
  return v.reshape(1, -1)


def kernel(h, x, edge_attr, t, params, edge_index):
  hpad = jnp.pad(h, ((0, N_PAD - N), (0, 0)))
  xtab = jnp.pad(x, ((0, N_PAD - N), (0, 128 - 3)))
  eapad = jnp.pad(edge_attr, ((0, E_PAD - E), (0, 0)))
  npad_e = E_PAD - E
  pad_idx = N + (jnp.arange(npad_e, dtype=jnp.int32) % 64)
  rflat = jnp.concatenate([edge_index[0], pad_idx])
  cflat = jnp.concatenate([edge_index[1], pad_idx])
  ri64 = rflat.reshape(NW, NG, KG)
  ci64 = cflat.reshape(NW, NG, KG)
  ri128 = rflat.reshape(NW, NCH, KCH)
  zer = jnp.zeros((N_PAD, 128), _f32)

  neb = _row(params['ne_b']) + _time_row(t, params)
  l0 = params['layers'][0]
  hn, pa, pb = _init_call(
      hpad, params['ne_w'], neb, _row(l0['ln_g']), _row(l0['ln_b']),
      l0['e_w1'][:D], l0['e_w1'][D:2 * D])

  out = None
  for li in range(4):
    lp = params['layers'][li]
    ga, gb = _gather_call(pa, pb, ri64, ci64)
    xr, xc = _gatherx_call(xtab, ri64, ci64)
    m2, cu = _edge_call(
        ga, gb, xr, xc, eapad,
        lp['e_w1'][2 * D + 1:], lp['e_w1'][2 * D:2 * D + 1], _row(lp['e_b1']),
        lp['e_w2'], _row(lp['e_b2']),
        _row(lp['a_w'][:, 0]), jnp.full((1, D), lp['a_b'][0] / D, _f32),
        lp['c_w1'], _row(lp['c_b1']), _row(lp['c_w2'][:, 0]))
    mi = _scatterm_call(m2, ri128, zer)
    xp = _scatterx_call(cu, ri64, zer)
    if li < 3:
      lnx = params['layers'][li + 1]
      hn, pa, pb, xtab = _node_call(
          hn, mi, xp, xtab,
          lp['n_w1'][:D], lp['n_w1'][D:D + 128], lp['n_w1'][D + 128:],
          _row(lp['n_b1']), lp['n_w2'], _row(lp['n_b2']),
          _row(lnx['ln_g']), _row(lnx['ln_b']),
          lnx['e_w1'][:D], lnx['e_w1'][D:2 * D])
    else:
      chw = jnp.pad(params['ch_w'], ((0, 0), (0, 128 - 3)))
      chb = jnp.pad(params['ch_b'], (0, 128 - 3)).reshape(1, 128)
      x0tab = jnp.pad(x, ((0, N_PAD - N), (0, 128 - 3)))
      ho, xo = _final_call(
          hn, mi, xp, xtab,
          lp['n_w1'][:D], lp['n_w1'][D:D + 128], lp['n_w1'][D + 128:],
          _row(lp['n_b1']), lp['n_w2'], _row(lp['n_b2']),
          params['o1_w'], _row(params['o1_b']),
          params['o2_w'], _row(params['o2_b']),
          chw, chb, x0tab)
      out = (ho[:N], xo[:N, :3])
  return out


# speedup vs baseline: 2.3650x; 1.0134x over previous
import functools
import math

import jax
import jax.numpy as jnp
from jax import lax
from jax.experimental import pallas as pl
from jax.experimental.pallas import tpu as pltpu
from jax.experimental.pallas import tpu_sc as plsc

N = 10000
E = 160000
D = 256
ED = 16
TE = 64

N_PAD = 10240
E_PAD = 163840
NW = 32
EPW = E_PAD // NW
KCH = 128
NCH = EPW // KCH
NSUB = 16
ROWS_PER_TILE = N_PAD // NSUB
XW = 16

BN = 1024
BE = 512

_f32 = jnp.float32


def _silu(z):
  return z * jax.nn.sigmoid(z)


def _ln(h, g, b):
  mu = jnp.mean(h, axis=-1, keepdims=True)
  v = jnp.mean((h - mu) * (h - mu), axis=-1, keepdims=True)
  return (h - mu) * lax.rsqrt(v + 1e-5) * g + b


def _init_body(h_ref, new_ref, neb_ref, g_ref, b_ref, wa_ref, wb_ref,
               hn_ref, pa_ref, pb_ref):
  h0 = jnp.dot(h_ref[...], new_ref[...], preferred_element_type=_f32)
  h0 = h0 + neb_ref[...]
  hn = _ln(h0, g_ref[...], b_ref[...])
  hn_ref[...] = hn
  pa_ref[...] = jnp.dot(hn, wa_ref[...], preferred_element_type=_f32)
  pb_ref[...] = jnp.dot(hn, wb_ref[...], preferred_element_type=_f32)


def _edge_body(ga_ref, gb_ref, xr_ref, xc_ref, ea_ref, wea_ref, wr_ref,
               b1_ref, ew2_ref, b2_ref, aw_ref, ab_ref, cw1_ref, cb1_ref,
               cw2_ref, m_ref, cu_ref):
  cd = xr_ref[...] - xc_ref[...]
  radial = jnp.sum(cd * cd, axis=1, keepdims=True)
  e_in = (ga_ref[...] + gb_ref[...] + radial * wr_ref[...]
          + jnp.dot(ea_ref[...], wea_ref[...], preferred_element_type=_f32)
          + b1_ref[...])
  m1 = _silu(e_in)
  m2 = _silu(jnp.dot(m1, ew2_ref[...], preferred_element_type=_f32)
             + b2_ref[...])
  att = jax.nn.sigmoid(
      jnp.sum(m2 * aw_ref[...] + ab_ref[...], axis=1, keepdims=True))
  m = m2 * att
  t1 = _silu(jnp.dot(m, cw1_ref[...], preferred_element_type=_f32)
             + cb1_ref[...])
  cw = jnp.sum(t1 * cw2_ref[...], axis=1, keepdims=True)
  inv = lax.rsqrt(radial + 1e-8) * cw
  cu_ref[0] = cd * inv
  m_ref[0] = m[:, :128]
  m_ref[1] = m[:, 128:]


def _node_body(hn_ref, mi_ref, w1a_ref, w1bl_ref, w1bh_ref,
               nb1_ref, nw2_ref, nb2_ref, g_ref, b_ref, wa_ref, wb_ref,
               hn_out_ref, pa_ref, pb_ref):
  z = (jnp.dot(hn_ref[...], w1a_ref[...], preferred_element_type=_f32)
       + jnp.dot(mi_ref[0], w1bl_ref[...], preferred_element_type=_f32)
       + jnp.dot(mi_ref[1], w1bh_ref[...], preferred_element_type=_f32)
       + nb1_ref[...])
  hnew = hn_ref[...] + jnp.dot(_silu(z), nw2_ref[...],
                               preferred_element_type=_f32) + nb2_ref[...]
  hn2 = _ln(hnew, g_ref[...], b_ref[...])
  hn_out_ref[...] = hn2
  pa_ref[...] = jnp.dot(hn2, wa_ref[...], preferred_element_type=_f32)
  pb_ref[...] = jnp.dot(hn2, wb_ref[...], preferred_element_type=_f32)


def _nodex_body(xt_ref, xp_ref, xt_out_ref):
  xt_out_ref[...] = xt_ref[...] + xp_ref[0] + xp_ref[1]


def _final_body(hn_ref, mi_ref, xp_ref, xt_ref, w1a_ref, w1bl_ref, w1bh_ref,
                nb1_ref, nw2_ref, nb2_ref, o1_ref, o1b_ref, o2_ref, o2b_ref,
                chw_ref, chb_ref, x0_ref, ho_ref, xo_ref):
  z = (jnp.dot(hn_ref[...], w1a_ref[...], preferred_element_type=_f32)
       + jnp.dot(mi_ref[0], w1bl_ref[...], preferred_element_type=_f32)
       + jnp.dot(mi_ref[1], w1bh_ref[...], preferred_element_type=_f32)
       + nb1_ref[...])
  h4 = hn_ref[...] + jnp.dot(_silu(z), nw2_ref[...],
                             preferred_element_type=_f32) + nb2_ref[...]
  ho = _silu(jnp.dot(h4, o1_ref[...], preferred_element_type=_f32)
             + o1b_ref[...])
  ho_ref[...] = jnp.dot(ho, o2_ref[...], preferred_element_type=_f32) \
      + o2b_ref[...]
  x4 = xt_ref[...] + xp_ref[0] + xp_ref[1]
  ch = jnp.dot(h4, chw_ref[...], preferred_element_type=_f32) + chb_ref[...]
  xo_ref[...] = x4 - x0_ref[...] + ch


def _row_spec(bn, w):
  return pl.BlockSpec((bn, w), lambda i: (i, 0))


def _full_spec(shape):
  nd = len(shape)
  return pl.BlockSpec(shape, lambda i: (0,) * nd)


_NODE_GRID = (N_PAD // BN,)
_EDGE_GRID = (E_PAD // BE,)
_TC_PARAMS = pltpu.CompilerParams(dimension_semantics=("arbitrary",))

_init_call = pl.pallas_call(
    _init_body,
    grid=_NODE_GRID,
    in_specs=[_row_spec(BN, D)] + [_full_spec((D, D))] + [_full_spec((1, D))] * 3
    + [_full_spec((D, D))] * 2,
    out_specs=[_row_spec(BN, D)] * 3,
    out_shape=[jax.ShapeDtypeStruct((N_PAD, D), _f32)] * 3,
    compiler_params=_TC_PARAMS,
)

_edge_call = pl.pallas_call(
    _edge_body,
    grid=_EDGE_GRID,
    in_specs=[
        _row_spec(BE, D), _row_spec(BE, D),
        _row_spec(BE, XW), _row_spec(BE, XW),
        _row_spec(BE, ED), _full_spec((ED, D)),
        _full_spec((1, D)), _full_spec((1, D)),
        _full_spec((D, D)), _full_spec((1, D)),
        _full_spec((1, D)), _full_spec((1, D)),
        _full_spec((D, D)), _full_spec((1, D)),
        _full_spec((1, D)),
    ],
    out_specs=[
        pl.BlockSpec((2, BE, 128), lambda i: (0, i, 0)),
        pl.BlockSpec((1, BE, XW), lambda i: (0, i, 0)),
    ],
    out_shape=[
        jax.ShapeDtypeStruct((2, E_PAD, 128), _f32),
        jax.ShapeDtypeStruct((1, E_PAD, XW), _f32),
    ],
    compiler_params=_TC_PARAMS,
)

_node_call = pl.pallas_call(
    _node_body,
    grid=_NODE_GRID,
    in_specs=[
        _row_spec(BN, D),
        pl.BlockSpec((2, BN, 128), lambda i: (0, i, 0)),
        _full_spec((D, D)), _full_spec((128, D)), _full_spec((128, D)),
        _full_spec((1, D)), _full_spec((D, D)), _full_spec((1, D)),
        _full_spec((1, D)), _full_spec((1, D)),
        _full_spec((D, D)), _full_spec((D, D)),
    ],
    out_specs=[_row_spec(BN, D)] * 3,
    out_shape=[jax.ShapeDtypeStruct((N_PAD, D), _f32)] * 3,
    compiler_params=_TC_PARAMS,
)

_nodex_call = pl.pallas_call(
    _nodex_body,
    grid=_NODE_GRID,
    in_specs=[
        _row_spec(BN, 128),
        pl.BlockSpec((2, BN, 128), lambda i: (0, i, 0)),
    ],
    out_specs=_row_spec(BN, 128),
    out_shape=jax.ShapeDtypeStruct((N_PAD, 128), _f32),
    compiler_params=_TC_PARAMS,
)

_final_call = pl.pallas_call(
    _final_body,
    grid=_NODE_GRID,
    in_specs=[
        _row_spec(BN, D),
        pl.BlockSpec((2, BN, 128), lambda i: (0, i, 0)),
        pl.BlockSpec((2, BN, 128), lambda i: (0, i, 0)),
        _row_spec(BN, 128),
        _full_spec((D, D)), _full_spec((128, D)), _full_spec((128, D)),
        _full_spec((1, D)), _full_spec((D, D)), _full_spec((1, D)),
        _full_spec((D, D)), _full_spec((1, D)),
        _full_spec((D, D)), _full_spec((1, D)),
        _full_spec((D, 128)), _full_spec((1, 128)),
        _row_spec(BN, 128),
    ],
    out_specs=[_row_spec(BN, D), _row_spec(BN, 128)],
    out_shape=[
        jax.ShapeDtypeStruct((N_PAD, D), _f32),
        jax.ShapeDtypeStruct((N_PAD, 128), _f32),
    ],
    compiler_params=_TC_PARAMS,
)


_SC_MESH = plsc.VectorSubcoreMesh(
    core_axis_name="c", subcore_axis_name="s", num_cores=2, num_subcores=NSUB)


KG = 64
NG = EPW // KG


@functools.partial(
    pl.kernel,
    out_type=[
        jax.ShapeDtypeStruct((E_PAD, D), _f32),
        jax.ShapeDtypeStruct((E_PAD, D), _f32),
        jax.ShapeDtypeStruct((E_PAD, XW), _f32),
        jax.ShapeDtypeStruct((E_PAD, XW), _f32),
    ],
    mesh=_SC_MESH,
    scratch_types=[
        pltpu.VMEM((NG, KG), jnp.int32),
        pltpu.VMEM((NG, KG), jnp.int32),
        [pltpu.VMEM((KG, D), _f32)] * 2,
        [pltpu.VMEM((KG, D), _f32)] * 2,
        pltpu.VMEM((KG, 128), _f32),
        pltpu.VMEM((KG, 128), _f32),
        pltpu.VMEM((KG, XW), _f32),
        pltpu.VMEM((KG, XW), _f32),
        [pltpu.SemaphoreType.DMA] * 2,
        [pltpu.SemaphoreType.DMA] * 2,
        pltpu.SemaphoreType.DMA,
        pltpu.SemaphoreType.DMA,
    ],
)
def _gather_call(pa_hbm, pb_hbm, xt_hbm, ri_hbm, ci_hbm,
                 ga_hbm, gb_hbm, xr_hbm, xc_hbm,
                 ri_v, ci_v, ba, bb, bxr, bxc, bxrs, bxcs, sa, sb, sr, sc_):
  c = lax.axis_index("c")
  s = lax.axis_index("s")
  wid = s * 2 + c
  pltpu.sync_copy(ri_hbm.at[wid], ri_v)
  pltpu.sync_copy(ci_hbm.at[wid], ci_v)
  base = wid * EPW

  def startf(j, k):
    pltpu.async_copy(pa_hbm.at[ri_v.at[j]], ba[k], sa[k])
    pltpu.async_copy(pb_hbm.at[ci_v.at[j]], bb[k], sb[k])

  def finishf(j, k):
    pltpu.make_async_copy(pa_hbm.at[ri_v.at[j]], ba[k], sa[k]).wait()
    pltpu.make_async_copy(pb_hbm.at[ci_v.at[j]], bb[k], sb[k]).wait()
    e0 = base + j * KG
    pltpu.sync_copy(ba[k], ga_hbm.at[pl.ds(e0, KG)])
    pltpu.sync_copy(bb[k], gb_hbm.at[pl.ds(e0, KG)])

  def startx(j):
    pltpu.async_copy(xt_hbm.at[ri_v.at[j]], bxr, sr)
    pltpu.async_copy(xt_hbm.at[ci_v.at[j]], bxc, sc_)

  def finishx(j):
    pltpu.make_async_copy(xt_hbm.at[ri_v.at[j]], bxr, sr).wait()
    pltpu.make_async_copy(xt_hbm.at[ci_v.at[j]], bxc, sc_).wait()

    def comp(i, cc):
      bxrs[i, :] = bxr[i, pl.ds(0, XW)]
      bxcs[i, :] = bxc[i, pl.ds(0, XW)]
      return cc

    lax.fori_loop(0, KG, comp, 0)
    e0 = base + j * KG
    pltpu.sync_copy(bxrs, xr_hbm.at[pl.ds(e0, KG)])
    pltpu.sync_copy(bxcs, xc_hbm.at[pl.ds(e0, KG)])

  startf(0, 0)
  startx(0)

  def body(g, carry):
    j0 = g * 2
    startf(j0 + 1, 1)
    finishf(j0, 0)
    finishx(j0)
    startx(j0 + 1)

    @pl.when(g < NG // 2 - 1)
    def _():
      startf(j0 + 2, 0)

    finishf(j0 + 1, 1)
    finishx(j0 + 1)

    @pl.when(g < NG // 2 - 1)
    def _():
      startx(j0 + 2)

    return carry

  lax.fori_loop(0, NG // 2, body, 0)


@functools.partial(
    pl.kernel,
    out_type=jax.ShapeDtypeStruct((2, N_PAD, 128), _f32),
    mesh=_SC_MESH,
    scratch_types=[
        pltpu.VMEM((2 * NCH, KCH), jnp.int32),
        [pltpu.VMEM((KCH, 128), _f32)] * 2,
        [pltpu.SemaphoreType.DMA] * 2,
        pltpu.VMEM_SHARED((N_PAD, 128), _f32),
    ],
)
def _scatterm_call(m2_hbm, ri_hbm, zer_hbm, mi_hbm, ri_v, bm, sm, acc_m):
  c = lax.axis_index("c")
  s = lax.axis_index("s")
  r0 = s * ROWS_PER_TILE
  pltpu.sync_copy(zer_hbm.at[pl.ds(r0, ROWS_PER_TILE)],
                  acc_m.at[pl.ds(r0, ROWS_PER_TILE)])
  plsc.subcore_barrier()
  pltpu.sync_copy(ri_hbm.at[s * 2], ri_v.at[pl.ds(0, NCH)])
  pltpu.sync_copy(ri_hbm.at[s * 2 + 1], ri_v.at[pl.ds(NCH, NCH)])
  base = s * 2 * EPW

  def start(j, k):
    pltpu.async_copy(m2_hbm.at[c, pl.ds(base + j * KCH, KCH)], bm[k], sm[k])

  def finish(j, k):
    pltpu.make_async_copy(
        m2_hbm.at[c, pl.ds(base + j * KCH, KCH)], bm[k], sm[k]).wait()
    pltpu.sync_copy(bm[k], acc_m.at[ri_v.at[j]], add=True)

  start(0, 0)

  def body(g, carry):
    j0 = g * 2
    start(j0 + 1, 1)
    finish(j0, 0)

    @pl.when(g < NCH - 1)
    def _():
      start(j0 + 2, 0)

    finish(j0 + 1, 1)
    return carry

  lax.fori_loop(0, NCH, body, 0)
  plsc.subcore_barrier()
  pltpu.sync_copy(acc_m.at[pl.ds(r0, ROWS_PER_TILE)],
                  mi_hbm.at[c, pl.ds(r0, ROWS_PER_TILE)])


@functools.partial(
    pl.kernel,
    out_type=jax.ShapeDtypeStruct((2, N_PAD, 128), _f32),
    mesh=_SC_MESH,
    scratch_types=[
        pltpu.VMEM((NG, KG), jnp.int32),
        [pltpu.VMEM((KG, XW), _f32)] * 2,
        pltpu.VMEM((KG, 128), _f32),
        [pltpu.SemaphoreType.DMA] * 2,
        pltpu.VMEM_SHARED((N_PAD, 128), _f32),
    ],
)
def _scatterx_call(cu_hbm, ri_hbm, zer_hbm, xp_hbm, ri_v, bcu, bcw, scu,
                   acc_x):
  c = lax.axis_index("c")
  s = lax.axis_index("s")
  r0 = s * ROWS_PER_TILE
  pltpu.sync_copy(zer_hbm.at[pl.ds(r0, ROWS_PER_TILE)],
                  acc_x.at[pl.ds(r0, ROWS_PER_TILE)])

  def zrow(i, cc):
    bcw[lax.div(i, 8), pl.ds(lax.rem(i, 8) * XW, XW)] = jnp.zeros((XW,), _f32)
    return cc

  lax.fori_loop(0, KG * 8, zrow, 0)
  plsc.subcore_barrier()
  wid = c * NSUB + s
  pltpu.sync_copy(ri_hbm.at[wid], ri_v)
  base = wid * EPW

  def start(j, k):
    pltpu.async_copy(cu_hbm.at[0, pl.ds(base + j * KG, KG)], bcu[k], scu[k])

  def finish(j, k):
    pltpu.make_async_copy(cu_hbm.at[0, pl.ds(base + j * KG, KG)],
                          bcu[k], scu[k]).wait()

    def widen(i, cc, k=k):
      bcw[i, pl.ds(0, XW)] = bcu[k][i, :]
      return cc

    lax.fori_loop(0, KG, widen, 0)
    pltpu.sync_copy(bcw, acc_x.at[ri_v.at[j]], add=True)

  start(0, 0)

  def body(g, carry):
    j0 = g * 2
    start(j0 + 1, 1)
    finish(j0, 0)

    @pl.when(g < NG // 2 - 1)
    def _():
      start(j0 + 2, 0)

    finish(j0 + 1, 1)
    return carry

  lax.fori_loop(0, NG // 2, body, 0)
  plsc.subcore_barrier()
  pltpu.sync_copy(acc_x.at[pl.ds(r0, ROWS_PER_TILE)],
                  xp_hbm.at[c, pl.ds(r0, ROWS_PER_TILE)])


def _time_row(t, params):
  half = TE // 2
  e = math.log(10000.0) / (half - 1)
  freqs = jnp.exp(jnp.arange(half, dtype=_f32) * -e)
  a = t[:, None] * freqs[None, :]
  te = jnp.concatenate([jnp.sin(a), jnp.cos(a)], axis=-1)
  te = _silu(te @ params['t1_w'] + params['t1_b'])
  te = te @ params['t2_w'] + params['t2_b']
  return te[0:1]


def _row(v):
  return v.reshape(1, -1)


def kernel(h, x, edge_attr, t, params, edge_index):
  hpad = jnp.pad(h, ((0, N_PAD - N), (0, 0)))
  xtab = jnp.pad(x, ((0, N_PAD - N), (0, 128 - 3)))
  eapad = jnp.pad(edge_attr, ((0, E_PAD - E), (0, 0)))
  npad_e = E_PAD - E
  pad_idx = N + (jnp.arange(npad_e, dtype=jnp.int32) % 64)
  rflat = jnp.concatenate([edge_index[0], pad_idx])
  cflat = jnp.concatenate([edge_index[1], pad_idx])
  ri64 = rflat.reshape(NW, NG, KG)
  ci64 = cflat.reshape(NW, NG, KG)
  ri128 = rflat.reshape(NW, NCH, KCH)
  zer = jnp.zeros((N_PAD, 128), _f32)

  neb = _row(params['ne_b']) + _time_row(t, params)
  l0 = params['layers'][0]
  hn, pa, pb = _init_call(
      hpad, params['ne_w'], neb, _row(l0['ln_g']), _row(l0['ln_b']),
      l0['e_w1'][:D], l0['e_w1'][D:2 * D])

  out = None
  for li in range(4):
    lp = params['layers'][li]
    ga, gb, xr, xc = _gather_call(pa, pb, xtab, ri64, ci64)
    m2, cu = _edge_call(
        ga, gb, xr, xc, eapad,
        lp['e_w1'][2 * D + 1:], lp['e_w1'][2 * D:2 * D + 1], _row(lp['e_b1']),
        lp['e_w2'], _row(lp['e_b2']),
        _row(lp['a_w'][:, 0]), jnp.full((1, D), lp['a_b'][0] / D, _f32),
        lp['c_w1'], _row(lp['c_b1']), _row(lp['c_w2'][:, 0]))
    mi = _scatterm_call(m2, ri128, zer)
    xp = _scatterx_call(cu, ri64, zer)
    if li < 3:
      lnx = params['layers'][li + 1]
      hn, pa, pb = _node_call(
          hn, mi,
          lp['n_w1'][:D], lp['n_w1'][D:D + 128], lp['n_w1'][D + 128:],
          _row(lp['n_b1']), lp['n_w2'], _row(lp['n_b2']),
          _row(lnx['ln_g']), _row(lnx['ln_b']),
          lnx['e_w1'][:D], lnx['e_w1'][D:2 * D])
      xtab = _nodex_call(xtab, xp)
    else:
      chw = jnp.pad(params['ch_w'], ((0, 0), (0, 128 - 3)))
      chb = jnp.pad(params['ch_b'], (0, 128 - 3)).reshape(1, 128)
      x0tab = jnp.pad(x, ((0, N_PAD - N), (0, 128 - 3)))
      ho, xo = _final_call(
          hn, mi, xp, xtab,
          lp['n_w1'][:D], lp['n_w1'][D:D + 128], lp['n_w1'][D + 128:],
          _row(lp['n_b1']), lp['n_w2'], _row(lp['n_b2']),
          params['o1_w'], _row(params['o1_b']),
          params['o2_w'], _row(params['o2_b']),
          chw, chb, x0tab)
      out = (ho[:N], xo[:N, :3])
  return out


# speedup vs baseline: 2.7837x; 1.1770x over previous
import functools
import math

import jax
import jax.numpy as jnp
from jax import lax
from jax.experimental import pallas as pl
from jax.experimental.pallas import tpu as pltpu
from jax.experimental.pallas import tpu_sc as plsc

N = 10000
E = 160000
D = 256
ED = 16
TE = 64

N_PAD = 10240
E_PAD = 163840
EH = E_PAD // 2
NW = 32
EPW = EH // NW
KCH = 128
NCH = EPW // KCH
NSUB = 16
ROWS_PER_TILE = N_PAD // NSUB
XW = 16

BN = 1024
BE = 1024

_f32 = jnp.float32


def _silu(z):
  return z * jax.nn.sigmoid(z)


def _ln(h, g, b):
  mu = jnp.mean(h, axis=-1, keepdims=True)
  v = jnp.mean((h - mu) * (h - mu), axis=-1, keepdims=True)
  return (h - mu) * lax.rsqrt(v + 1e-5) * g + b


def _init_body(h_ref, new_ref, neb_ref, g_ref, b_ref, wa_ref, wb_ref,
               hn_ref, pa_ref, pb_ref):
  h0 = jnp.dot(h_ref[...], new_ref[...], preferred_element_type=_f32)
  h0 = h0 + neb_ref[...]
  hn = _ln(h0, g_ref[...], b_ref[...])
  hn_ref[...] = hn
  pa_ref[...] = jnp.dot(hn, wa_ref[...], preferred_element_type=_f32)
  pb_ref[...] = jnp.dot(hn, wb_ref[...], preferred_element_type=_f32)


def _edge_body(ga_ref, gb_ref, xr_ref, xc_ref, ea_ref, wea_ref, wr_ref,
               b1_ref, ew2_ref, b2_ref, aw_ref, ab_ref, cw1_ref, cb1_ref,
               cw2_ref, m_ref, cu_ref):
  cd = xr_ref[...] - xc_ref[...]
  radial = jnp.sum(cd * cd, axis=1, keepdims=True)
  e_in = (ga_ref[...] + gb_ref[...] + radial * wr_ref[...]
          + jnp.dot(ea_ref[...], wea_ref[...], preferred_element_type=_f32)
          + b1_ref[...])
  m1 = _silu(e_in)
  m2 = _silu(jnp.dot(m1, ew2_ref[...], preferred_element_type=_f32)
             + b2_ref[...])
  att = jax.nn.sigmoid(
      jnp.sum(m2 * aw_ref[...] + ab_ref[...], axis=1, keepdims=True))
  m = m2 * att
  t1 = _silu(jnp.dot(m, cw1_ref[...], preferred_element_type=_f32)
             + cb1_ref[...])
  cw = jnp.sum(t1 * cw2_ref[...], axis=1, keepdims=True)
  inv = lax.rsqrt(radial + 1e-8) * cw
  cu_ref[0] = cd * inv
  m_ref[0] = m[:, :128]
  m_ref[1] = m[:, 128:]


def _node_body(hn_ref, mia_ref, mib_ref, w1a_ref, w1bl_ref, w1bh_ref,
               nb1_ref, nw2_ref, nb2_ref, g_ref, b_ref, wa_ref, wb_ref,
               hn_out_ref, pa_ref, pb_ref):
  z = (jnp.dot(hn_ref[...], w1a_ref[...], preferred_element_type=_f32)
       + jnp.dot(mia_ref[0] + mib_ref[0], w1bl_ref[...],
                 preferred_element_type=_f32)
       + jnp.dot(mia_ref[1] + mib_ref[1], w1bh_ref[...],
                 preferred_element_type=_f32)
       + nb1_ref[...])
  hnew = hn_ref[...] + jnp.dot(_silu(z), nw2_ref[...],
                               preferred_element_type=_f32) + nb2_ref[...]
  hn2 = _ln(hnew, g_ref[...], b_ref[...])
  hn_out_ref[...] = hn2
  pa_ref[...] = jnp.dot(hn2, wa_ref[...], preferred_element_type=_f32)
  pb_ref[...] = jnp.dot(hn2, wb_ref[...], preferred_element_type=_f32)


def _nodex_body(xt_ref, xpa_ref, xpb_ref, xt_out_ref):
  xt_out_ref[...] = (xt_ref[...] + xpa_ref[0] + xpa_ref[1]
                     + xpb_ref[0] + xpb_ref[1])


def _final_body(hn_ref, mia_ref, mib_ref, xpa_ref, xpb_ref, xt_ref,
                w1a_ref, w1bl_ref, w1bh_ref,
                nb1_ref, nw2_ref, nb2_ref, o1_ref, o1b_ref, o2_ref, o2b_ref,
                chw_ref, chb_ref, x0_ref, ho_ref, xo_ref):
  z = (jnp.dot(hn_ref[...], w1a_ref[...], preferred_element_type=_f32)
       + jnp.dot(mia_ref[0] + mib_ref[0], w1bl_ref[...],
                 preferred_element_type=_f32)
       + jnp.dot(mia_ref[1] + mib_ref[1], w1bh_ref[...],
                 preferred_element_type=_f32)
       + nb1_ref[...])
  h4 = hn_ref[...] + jnp.dot(_silu(z), nw2_ref[...],
                             preferred_element_type=_f32) + nb2_ref[...]
  ho = _silu(jnp.dot(h4, o1_ref[...], preferred_element_type=_f32)
             + o1b_ref[...])
  ho_ref[...] = jnp.dot(ho, o2_ref[...], preferred_element_type=_f32) \
      + o2b_ref[...]
  x4 = (xt_ref[...] + xpa_ref[0] + xpa_ref[1]
        + xpb_ref[0] + xpb_ref[1])
  ch = jnp.dot(h4, chw_ref[...], preferred_element_type=_f32) + chb_ref[...]
  xo_ref[...] = x4 - x0_ref[...] + ch


def _row_spec(bn, w):
  return pl.BlockSpec((bn, w), lambda i: (i, 0))


def _full_spec(shape):
  nd = len(shape)
  return pl.BlockSpec(shape, lambda i: (0,) * nd)


_NODE_GRID = (N_PAD // BN,)
_EDGE_GRID = (EH // BE,)
_TC_PARAMS = pltpu.CompilerParams(dimension_semantics=("arbitrary",))

_init_call = pl.pallas_call(
    _init_body,
    grid=_NODE_GRID,
    in_specs=[_row_spec(BN, D)] + [_full_spec((D, D))] + [_full_spec((1, D))] * 3
    + [_full_spec((D, D))] * 2,
    out_specs=[_row_spec(BN, D)] * 3,
    out_shape=[jax.ShapeDtypeStruct((N_PAD, D), _f32)] * 3,
    compiler_params=_TC_PARAMS,
)

_edge_call = pl.pallas_call(
    _edge_body,
    grid=_EDGE_GRID,
    in_specs=[
        _row_spec(BE, D), _row_spec(BE, D),
        _row_spec(BE, XW), _row_spec(BE, XW),
        _row_spec(BE, ED), _full_spec((ED, D)),
        _full_spec((1, D)), _full_spec((1, D)),
        _full_spec((D, D)), _full_spec((1, D)),
        _full_spec((1, D)), _full_spec((1, D)),
        _full_spec((D, D)), _full_spec((1, D)),
        _full_spec((1, D)),
    ],
    out_specs=[
        pl.BlockSpec((2, BE, 128), lambda i: (0, i, 0)),
        pl.BlockSpec((1, BE, XW), lambda i: (0, i, 0)),
    ],
    out_shape=[
        jax.ShapeDtypeStruct((2, EH, 128), _f32),
        jax.ShapeDtypeStruct((1, EH, XW), _f32),
    ],
    compiler_params=_TC_PARAMS,
)

_node_call = pl.pallas_call(
    _node_body,
    grid=_NODE_GRID,
    in_specs=[
        _row_spec(BN, D),
        pl.BlockSpec((2, BN, 128), lambda i: (0, i, 0)),
        pl.BlockSpec((2, BN, 128), lambda i: (0, i, 0)),
        _full_spec((D, D)), _full_spec((128, D)), _full_spec((128, D)),
        _full_spec((1, D)), _full_spec((D, D)), _full_spec((1, D)),
        _full_spec((1, D)), _full_spec((1, D)),
        _full_spec((D, D)), _full_spec((D, D)),
    ],
    out_specs=[_row_spec(BN, D)] * 3,
    out_shape=[jax.ShapeDtypeStruct((N_PAD, D), _f32)] * 3,
    compiler_params=_TC_PARAMS,
)

_nodex_call = pl.pallas_call(
    _nodex_body,
    grid=_NODE_GRID,
    in_specs=[
        _row_spec(BN, 128),
        pl.BlockSpec((2, BN, 128), lambda i: (0, i, 0)),
        pl.BlockSpec((2, BN, 128), lambda i: (0, i, 0)),
    ],
    out_specs=_row_spec(BN, 128),
    out_shape=jax.ShapeDtypeStruct((N_PAD, 128), _f32),
    compiler_params=_TC_PARAMS,
)

_final_call = pl.pallas_call(
    _final_body,
    grid=_NODE_GRID,
    in_specs=[
        _row_spec(BN, D),
        pl.BlockSpec((2, BN, 128), lambda i: (0, i, 0)),
        pl.BlockSpec((2, BN, 128), lambda i: (0, i, 0)),
        pl.BlockSpec((2, BN, 128), lambda i: (0, i, 0)),
        pl.BlockSpec((2, BN, 128), lambda i: (0, i, 0)),
        _row_spec(BN, 128),
        _full_spec((D, D)), _full_spec((128, D)), _full_spec((128, D)),
        _full_spec((1, D)), _full_spec((D, D)), _full_spec((1, D)),
        _full_spec((D, D)), _full_spec((1, D)),
        _full_spec((D, D)), _full_spec((1, D)),
        _full_spec((D, 128)), _full_spec((1, 128)),
        _row_spec(BN, 128),
    ],
    out_specs=[_row_spec(BN, D), _row_spec(BN, 128)],
    out_shape=[
        jax.ShapeDtypeStruct((N_PAD, D), _f32),
        jax.ShapeDtypeStruct((N_PAD, 128), _f32),
    ],
    compiler_params=_TC_PARAMS,
)


_SC_MESH = plsc.VectorSubcoreMesh(
    core_axis_name="c", subcore_axis_name="s", num_cores=2, num_subcores=NSUB)


KG = 64
NG = EPW // KG


@functools.partial(
    pl.kernel,
    out_type=[
        jax.ShapeDtypeStruct((EH, D), _f32),
        jax.ShapeDtypeStruct((EH, D), _f32),
        jax.ShapeDtypeStruct((EH, XW), _f32),
        jax.ShapeDtypeStruct((EH, XW), _f32),
    ],
    mesh=_SC_MESH,
    scratch_types=[
        pltpu.VMEM((NG, KG), jnp.int32),
        pltpu.VMEM((NG, KG), jnp.int32),
        [pltpu.VMEM((KG, D), _f32)] * 2,
        [pltpu.VMEM((KG, D), _f32)] * 2,
        pltpu.VMEM((KG, 128), _f32),
        pltpu.VMEM((KG, 128), _f32),
        pltpu.VMEM((KG, XW), _f32),
        pltpu.VMEM((KG, XW), _f32),
        [pltpu.SemaphoreType.DMA] * 2,
        [pltpu.SemaphoreType.DMA] * 2,
        pltpu.SemaphoreType.DMA,
        pltpu.SemaphoreType.DMA,
    ],
)
def _gather_call(pa_hbm, pb_hbm, xt_hbm, ri_hbm, ci_hbm,
                 ga_hbm, gb_hbm, xr_hbm, xc_hbm,
                 ri_v, ci_v, ba, bb, bxr, bxc, bxrs, bxcs, sa, sb, sr, sc_):
  c = lax.axis_index("c")
  s = lax.axis_index("s")
  wid = s * 2 + c
  pltpu.sync_copy(ri_hbm.at[wid], ri_v)
  pltpu.sync_copy(ci_hbm.at[wid], ci_v)
  base = wid * EPW

  def startf(j, k):
    pltpu.async_copy(pa_hbm.at[ri_v.at[j]], ba[k], sa[k])
    pltpu.async_copy(pb_hbm.at[ci_v.at[j]], bb[k], sb[k])

  def finishf(j, k):
    pltpu.make_async_copy(pa_hbm.at[ri_v.at[j]], ba[k], sa[k]).wait()
    pltpu.make_async_copy(pb_hbm.at[ci_v.at[j]], bb[k], sb[k]).wait()
    e0 = base + j * KG
    pltpu.sync_copy(ba[k], ga_hbm.at[pl.ds(e0, KG)])
    pltpu.sync_copy(bb[k], gb_hbm.at[pl.ds(e0, KG)])

  def startx(j):
    pltpu.async_copy(xt_hbm.at[ri_v.at[j]], bxr, sr)
    pltpu.async_copy(xt_hbm.at[ci_v.at[j]], bxc, sc_)

  def finishx(j):
    pltpu.make_async_copy(xt_hbm.at[ri_v.at[j]], bxr, sr).wait()
    pltpu.make_async_copy(xt_hbm.at[ci_v.at[j]], bxc, sc_).wait()

    def comp(i, cc):
      bxrs[i, :] = bxr[i, pl.ds(0, XW)]
      bxcs[i, :] = bxc[i, pl.ds(0, XW)]
      return cc

    lax.fori_loop(0, KG, comp, 0)
    e0 = base + j * KG
    pltpu.sync_copy(bxrs, xr_hbm.at[pl.ds(e0, KG)])
    pltpu.sync_copy(bxcs, xc_hbm.at[pl.ds(e0, KG)])

  startf(0, 0)
  startx(0)

  def body(g, carry):
    j0 = g * 2
    startf(j0 + 1, 1)
    finishf(j0, 0)
    finishx(j0)
    startx(j0 + 1)

    @pl.when(g < NG // 2 - 1)
    def _():
      startf(j0 + 2, 0)

    finishf(j0 + 1, 1)
    finishx(j0 + 1)

    @pl.when(g < NG // 2 - 1)
    def _():
      startx(j0 + 2)

    return carry

  lax.fori_loop(0, NG // 2, body, 0)


@functools.partial(
    pl.kernel,
    out_type=jax.ShapeDtypeStruct((2, N_PAD, 128), _f32),
    mesh=_SC_MESH,
    scratch_types=[
        pltpu.VMEM((2 * NCH, KCH), jnp.int32),
        [pltpu.VMEM((KCH, 128), _f32)] * 2,
        [pltpu.SemaphoreType.DMA] * 2,
        pltpu.VMEM_SHARED((N_PAD, 128), _f32),
    ],
)
def _scatterm_call(m2_hbm, ri_hbm, zer_hbm, mi_hbm, ri_v, bm, sm, acc_m):
  c = lax.axis_index("c")
  s = lax.axis_index("s")
  r0 = s * ROWS_PER_TILE
  pltpu.sync_copy(zer_hbm.at[pl.ds(r0, ROWS_PER_TILE)],
                  acc_m.at[pl.ds(r0, ROWS_PER_TILE)])
  plsc.subcore_barrier()
  pltpu.sync_copy(ri_hbm.at[s * 2], ri_v.at[pl.ds(0, NCH)])
  pltpu.sync_copy(ri_hbm.at[s * 2 + 1], ri_v.at[pl.ds(NCH, NCH)])
  base = s * 2 * EPW

  def start(j, k):
    pltpu.async_copy(m2_hbm.at[c, pl.ds(base + j * KCH, KCH)], bm[k], sm[k])

  def finish(j, k):
    pltpu.make_async_copy(
        m2_hbm.at[c, pl.ds(base + j * KCH, KCH)], bm[k], sm[k]).wait()
    pltpu.sync_copy(bm[k], acc_m.at[ri_v.at[j]], add=True)

  start(0, 0)

  def body(g, carry):
    j0 = g * 2
    start(j0 + 1, 1)
    finish(j0, 0)

    @pl.when(g < NCH - 1)
    def _():
      start(j0 + 2, 0)

    finish(j0 + 1, 1)
    return carry

  lax.fori_loop(0, NCH, body, 0)
  plsc.subcore_barrier()
  pltpu.sync_copy(acc_m.at[pl.ds(r0, ROWS_PER_TILE)],
                  mi_hbm.at[c, pl.ds(r0, ROWS_PER_TILE)])


@functools.partial(
    pl.kernel,
    out_type=jax.ShapeDtypeStruct((2, N_PAD, 128), _f32),
    mesh=_SC_MESH,
    scratch_types=[
        pltpu.VMEM((NG, KG), jnp.int32),
        [pltpu.VMEM((KG, XW), _f32)] * 2,
        pltpu.VMEM((KG, 128), _f32),
        [pltpu.SemaphoreType.DMA] * 2,
        pltpu.VMEM_SHARED((N_PAD, 128), _f32),
    ],
)
def _scatterx_call(cu_hbm, ri_hbm, zer_hbm, xp_hbm, ri_v, bcu, bcw, scu,
                   acc_x):
  c = lax.axis_index("c")
  s = lax.axis_index("s")
  r0 = s * ROWS_PER_TILE
  pltpu.sync_copy(zer_hbm.at[pl.ds(r0, ROWS_PER_TILE)],
                  acc_x.at[pl.ds(r0, ROWS_PER_TILE)])

  def zrow(i, cc):
    bcw[lax.div(i, 8), pl.ds(lax.rem(i, 8) * XW, XW)] = jnp.zeros((XW,), _f32)
    return cc

  lax.fori_loop(0, KG * 8, zrow, 0)
  plsc.subcore_barrier()
  wid = c * NSUB + s
  pltpu.sync_copy(ri_hbm.at[wid], ri_v)
  base = wid * EPW

  def start(j, k):
    pltpu.async_copy(cu_hbm.at[0, pl.ds(base + j * KG, KG)], bcu[k], scu[k])

  def finish(j, k):
    pltpu.make_async_copy(cu_hbm.at[0, pl.ds(base + j * KG, KG)],
                          bcu[k], scu[k]).wait()

    def widen(i, cc, k=k):
      bcw[i, pl.ds(0, XW)] = bcu[k][i, :]
      return cc

    lax.fori_loop(0, KG, widen, 0)
    pltpu.sync_copy(bcw, acc_x.at[ri_v.at[j]], add=True)

  start(0, 0)

  def body(g, carry):
    j0 = g * 2
    start(j0 + 1, 1)
    finish(j0, 0)

    @pl.when(g < NG // 2 - 1)
    def _():
      start(j0 + 2, 0)

    finish(j0 + 1, 1)
    return carry

  lax.fori_loop(0, NG // 2, body, 0)
  plsc.subcore_barrier()
  pltpu.sync_copy(acc_x.at[pl.ds(r0, ROWS_PER_TILE)],
                  xp_hbm.at[c, pl.ds(r0, ROWS_PER_TILE)])


def _time_row(t, params):
  half = TE // 2
  e = math.log(10000.0) / (half - 1)
  freqs = jnp.exp(jnp.arange(half, dtype=_f32) * -e)
  a = t[:, None] * freqs[None, :]
  te = jnp.concatenate([jnp.sin(a), jnp.cos(a)], axis=-1)
  te = _silu(te @ params['t1_w'] + params['t1_b'])
  te = te @ params['t2_w'] + params['t2_b']
  return te[0:1]


def _row(v):
  return v.reshape(1, -1)


def kernel(h, x, edge_attr, t, params, edge_index):
  hpad = jnp.pad(h, ((0, N_PAD - N), (0, 0)))
  xtab = jnp.pad(x, ((0, N_PAD - N), (0, 128 - 3)))
  eapad = jnp.pad(edge_attr, ((0, E_PAD - E), (0, 0)))
  npad_e = E_PAD - E
  pad_idx = N + (jnp.arange(npad_e, dtype=jnp.int32) % 64)
  rflat = jnp.concatenate([edge_index[0], pad_idx])
  cflat = jnp.concatenate([edge_index[1], pad_idx])
  halves = []
  for hs in (slice(0, EH), slice(EH, E_PAD)):
    halves.append(dict(
        ri64=rflat[hs].reshape(NW, NG, KG),
        ci64=cflat[hs].reshape(NW, NG, KG),
        ri128=rflat[hs].reshape(NW, NCH, KCH),
        ea=eapad[hs]))
  zer = jnp.zeros((N_PAD, 128), _f32)

  neb = _row(params['ne_b']) + _time_row(t, params)
  l0 = params['layers'][0]
  hn, pa, pb = _init_call(
      hpad, params['ne_w'], neb, _row(l0['ln_g']), _row(l0['ln_b']),
      l0['e_w1'][:D], l0['e_w1'][D:2 * D])

  out = None
  for li in range(4):
    lp = params['layers'][li]
    ew = (lp['e_w1'][2 * D + 1:], lp['e_w1'][2 * D:2 * D + 1],
          _row(lp['e_b1']), lp['e_w2'], _row(lp['e_b2']),
          _row(lp['a_w'][:, 0]), jnp.full((1, D), lp['a_b'][0] / D, _f32),
          lp['c_w1'], _row(lp['c_b1']), _row(lp['c_w2'][:, 0]))
    mis = []
    xps = []
    gs = [_gather_call(pa, pb, xtab, hv['ri64'], hv['ci64']) for hv in halves]
    for hv, (ga, gb, xr, xc) in zip(halves, gs):
      m2, cu = _edge_call(ga, gb, xr, xc, hv['ea'], *ew)
      mis.append(_scatterm_call(m2, hv['ri128'], zer))
      xps.append(_scatterx_call(cu, hv['ri64'], zer))
    if li < 3:
      lnx = params['layers'][li + 1]
      hn, pa, pb = _node_call(
          hn, mis[0], mis[1],
          lp['n_w1'][:D], lp['n_w1'][D:D + 128], lp['n_w1'][D + 128:],
          _row(lp['n_b1']), lp['n_w2'], _row(lp['n_b2']),
          _row(lnx['ln_g']), _row(lnx['ln_b']),
          lnx['e_w1'][:D], lnx['e_w1'][D:2 * D])
      xtab = _nodex_call(xtab, xps[0], xps[1])
    else:
      chw = jnp.pad(params['ch_w'], ((0, 0), (0, 128 - 3)))
      chb = jnp.pad(params['ch_b'], (0, 128 - 3)).reshape(1, 128)
      x0tab = jnp.pad(x, ((0, N_PAD - N), (0, 128 - 3)))
      ho, xo = _final_call(
          hn, mis[0], mis[1], xps[0], xps[1], xtab,
          lp['n_w1'][:D], lp['n_w1'][D:D + 128], lp['n_w1'][D + 128:],
          _row(lp['n_b1']), lp['n_w2'], _row(lp['n_b2']),
          params['o1_w'], _row(params['o1_b']),
          params['o2_w'], _row(params['o2_b']),
          chw, chb, x0tab)
      out = (ho[:N], xo[:N, :3])
  return out
